# speedup vs baseline: 9.4975x; 9.4975x over previous
import functools

import jax
import jax.numpy as jnp
from jax import lax
from jax.experimental import pallas as pl
from jax.experimental.pallas import tpu as pltpu
from jax.experimental.pallas import tpu_sc as plsc

N = 10000
E = 320000
B = 64
EMB = 128
HID = 128
FP = 2048

NPAD = 10240
NC, NS = 2, 16
NW = NC * NS
EPW = E // NW
C = 80
NCHUNK = EPW // C
RPT = NPAD // NS
GCH = NPAD // C
R = 256
GRID = NPAD // R

_f32 = jnp.float32


def _mesh():
    return plsc.VectorSubcoreMesh(
        core_axis_name="c", subcore_axis_name="s", num_cores=NC, num_subcores=NS
    )


def _emb_body(f0, f1, f2, f3, t0, t1, t2, t3, out, idx_v, rows_v, sem):
    c = lax.axis_index("c")
    s = lax.axis_index("s")
    w = s * NC + c
    fields = (f0, f1, f2, f3)
    tables = (t0, t1, t2, t3)
    for f in range(4):
        for i in range(4):
            j = w + NW * i
            base = pl.multiple_of(j * C, 8)
            pltpu.sync_copy(fields[f].at[pl.ds(base, C)], idx_v)
            pltpu.async_copy(tables[f].at[idx_v], rows_v, sem).wait()
            pltpu.sync_copy(rows_v, out.at[f, pl.ds(base, C)])


def _deg_body(dst, zrow, deg_out, idx_d, ones_v, acc_sh, sem):
    c = lax.axis_index("c")
    s = lax.axis_index("s")
    w = s * NC + c
    for i in range(C // 16):
        ones_v[pl.ds(i * 16, 16)] = jnp.ones((16,), _f32)
    off = pl.multiple_of(s * RPT, 8)
    pltpu.sync_copy(zrow, acc_sh.at[pl.ds(off, RPT)])
    plsc.subcore_barrier()
    base = pl.multiple_of(w * EPW, 8)

    def body(j, carry):
        eb = pl.multiple_of(base + j * C, 8)
        pltpu.sync_copy(dst.at[pl.ds(eb, C)], idx_d)
        pltpu.sync_copy(ones_v, acc_sh.at[idx_d], add=True)
        return carry

    lax.fori_loop(0, NCHUNK, body, 0)
    plsc.subcore_barrier()
    pltpu.sync_copy(acc_sh.at[pl.ds(off, RPT)], deg_out.at[c, pl.ds(off, RPT)])


def _scat_body(g, src, dst, zrows, out, idx_s, idx_d, rows, acc_sh, sem):
    c = lax.axis_index("c")
    s = lax.axis_index("s")
    w = s * NC + c
    off = pl.multiple_of(s * RPT, 8)
    pltpu.sync_copy(zrows, acc_sh.at[pl.ds(off, RPT)])
    plsc.subcore_barrier()
    base = pl.multiple_of(w * EPW, 8)

    def body(j, carry):
        eb = pl.multiple_of(base + j * C, 8)
        pltpu.sync_copy(src.at[pl.ds(eb, C)], idx_s)
        pltpu.sync_copy(dst.at[pl.ds(eb, C)], idx_d)
        pltpu.async_copy(g.at[idx_s], rows, sem).wait()
        pltpu.sync_copy(rows, acc_sh.at[idx_d], add=True)
        return carry

    lax.fori_loop(0, NCHUNK, body, 0)
    plsc.subcore_barrier()
    pltpu.sync_copy(acc_sh.at[pl.ds(off, RPT)], out.at[c, pl.ds(off, RPT)])


def _sc_emb(fp0, fp1, fp2, fp3, e0, e1, e2, e3):
    fn = pl.kernel(
        _emb_body,
        out_type=jax.ShapeDtypeStruct((4, NPAD, EMB), _f32),
        mesh=_mesh(),
        scratch_types=[
            pltpu.VMEM((C,), jnp.int32),
            pltpu.VMEM((C, EMB), _f32),
            pltpu.SemaphoreType.DMA,
        ],
    )
    return fn(fp0, fp1, fp2, fp3, e0, e1, e2, e3)


def _sc_deg(dst, zrow):
    fn = pl.kernel(
        _deg_body,
        out_type=jax.ShapeDtypeStruct((NC, NPAD), _f32),
        mesh=_mesh(),
        scratch_types=[
            pltpu.VMEM((C,), jnp.int32),
            pltpu.VMEM((C,), _f32),
            pltpu.VMEM_SHARED((NPAD,), _f32),
            pltpu.SemaphoreType.DMA,
        ],
    )
    return fn(dst, zrow)


def _sc_scatter(g, src, dst, zrows):
    fn = pl.kernel(
        _scat_body,
        out_type=jax.ShapeDtypeStruct((NC, NPAD, HID), _f32),
        mesh=_mesh(),
        scratch_types=[
            pltpu.VMEM((C,), jnp.int32),
            pltpu.VMEM((C,), jnp.int32),
            pltpu.VMEM((C, HID), _f32),
            pltpu.VMEM_SHARED((NPAD, HID), _f32),
            pltpu.SemaphoreType.DMA,
        ],
    )
    return fn(g, src, dst, zrows)


def _dinv(deg_ref):
    d = deg_ref[0] + deg_ref[1] + 1.0
    return lax.rsqrt(d)


def _t0_body(embs_ref, pW1_ref, pb1_ref, pW2_ref, pb2_ref, gW0_ref, deg_ref,
             out_ref):
    h = jnp.dot(embs_ref[0], pW1_ref[0], preferred_element_type=_f32)
    for f in range(1, 4):
        h += jnp.dot(embs_ref[f], pW1_ref[f], preferred_element_type=_f32)
    h = jnp.maximum(h + pb1_ref[...], 0.0)
    h = jnp.dot(h, pW2_ref[...], preferred_element_type=_f32) + pb2_ref[...]
    u = jnp.dot(h, gW0_ref[...], preferred_element_type=_f32)
    out_ref[...] = _dinv(deg_ref) * u


def _tc_proj(embs, pW1r, pb1, pW2, pb2, gW0, degr):
    return pl.pallas_call(
        _t0_body,
        grid=(GRID,),
        in_specs=[
            pl.BlockSpec((4, R, EMB), lambda i: (0, i, 0)),
            pl.BlockSpec((4, EMB, EMB), lambda i: (0, 0, 0)),
            pl.BlockSpec((1, EMB), lambda i: (0, 0)),
            pl.BlockSpec((EMB, EMB), lambda i: (0, 0)),
            pl.BlockSpec((1, EMB), lambda i: (0, 0)),
            pl.BlockSpec((EMB, HID), lambda i: (0, 0)),
            pl.BlockSpec((2, R, 1), lambda i: (0, i, 0)),
        ],
        out_specs=pl.BlockSpec((R, HID), lambda i: (i, 0)),
        out_shape=jax.ShapeDtypeStruct((NPAD, HID), _f32),
    )(embs, pW1r, pb1, pW2, pb2, gW0, degr)


def _comb_body(acc_ref, g_ref, deg_ref, gb_ref, W_ref, out_ref):
    dinv = _dinv(deg_ref)
    x = dinv * (acc_ref[0] + acc_ref[1] + g_ref[...]) + gb_ref[...]
    x = jnp.maximum(x, 0.0)
    out_ref[...] = dinv * jnp.dot(x, W_ref[...], preferred_element_type=_f32)


def _tc_combine(acc, g, degr, gb, W):
    return pl.pallas_call(
        _comb_body,
        grid=(GRID,),
        in_specs=[
            pl.BlockSpec((2, R, HID), lambda i: (0, i, 0)),
            pl.BlockSpec((R, HID), lambda i: (i, 0)),
            pl.BlockSpec((2, R, 1), lambda i: (0, i, 0)),
            pl.BlockSpec((1, HID), lambda i: (0, 0)),
            pl.BlockSpec((HID, HID), lambda i: (0, 0)),
        ],
        out_specs=pl.BlockSpec((R, HID), lambda i: (i, 0)),
        out_shape=jax.ShapeDtypeStruct((NPAD, HID), _f32),
    )(acc, g, degr, gb, W)


def _pool_body(acc_ref, g_ref, deg_ref, gb_ref, batch_ref, out_ref):
    dinv = _dinv(deg_ref)
    x = dinv * (acc_ref[0] + acc_ref[1] + g_ref[...]) + gb_ref[...]
    x = jnp.maximum(x, 0.0)
    b = batch_ref[...]
    oh = (b == lax.broadcasted_iota(jnp.int32, (R, B), 1)).astype(_f32)
    part = lax.dot_general(oh, x, (((0,), (0,)), ((), ())),
                           preferred_element_type=_f32)

    @pl.when(pl.program_id(0) == 0)
    def _():
        out_ref[...] = part

    @pl.when(pl.program_id(0) != 0)
    def _():
        out_ref[...] += part


def _tc_pool(acc, g, degr, gb, batch_p):
    return pl.pallas_call(
        _pool_body,
        grid=(GRID,),
        in_specs=[
            pl.BlockSpec((2, R, HID), lambda i: (0, i, 0)),
            pl.BlockSpec((R, HID), lambda i: (i, 0)),
            pl.BlockSpec((2, R, 1), lambda i: (0, i, 0)),
            pl.BlockSpec((1, HID), lambda i: (0, 0)),
            pl.BlockSpec((R, 1), lambda i: (i, 0)),
        ],
        out_specs=pl.BlockSpec((B, HID), lambda i: (0, 0)),
        out_shape=jax.ShapeDtypeStruct((B, HID), _f32),
    )(acc, g, degr, gb, batch_p)


def _head_body(fp_ref, fpW_ref, fpb_ref, pooled_ref, l1a_ref, l1b_ref,
               l1bias_ref, l2W_ref, l2b_ref, out_ref):
    fpe = jnp.dot(fp_ref[...], fpW_ref[...], preferred_element_type=_f32)
    fpe = fpe + fpb_ref[...]
    z = (jnp.dot(fpe, l1a_ref[...], preferred_element_type=_f32)
         + jnp.dot(pooled_ref[...], l1b_ref[...], preferred_element_type=_f32)
         + l1bias_ref[...])
    z = jnp.maximum(z, 0.0)
    out_ref[...] = (jnp.dot(z, l2W_ref[...], preferred_element_type=_f32)
                    + l2b_ref[...])


def _tc_head(fingerprint, fpW, fpb, pooled, l1Wa, l1Wb, l1b, l2W, l2b):
    return pl.pallas_call(
        _head_body,
        out_shape=jax.ShapeDtypeStruct((B, 1), _f32),
    )(fingerprint, fpW, fpb, pooled, l1Wa, l1Wb, l1b, l2W, l2b)


def kernel(f0, f1, f2, f3, edge_index, batch, fingerprint,
           emb0, emb1, emb2, emb3, pW1, pb1, pW2, pb2,
           gW0, gb0, gW1, gb1, gW2, gb2, fpW, fpb, l1W, l1b, l2W, l2b):
    src = edge_index[0]
    dst = edge_index[1]

    pad_i = jnp.arange(NPAD - N, dtype=jnp.int32)
    fps = [jnp.concatenate([f.astype(jnp.int32), pad_i]) for f in (f0, f1, f2, f3)]
    batch_p = jnp.concatenate(
        [batch.astype(jnp.int32), jnp.full((NPAD - N,), B, jnp.int32)]
    ).reshape(NPAD, 1)
    zrow = jnp.zeros((RPT,), _f32)
    zrows = jnp.zeros((RPT, HID), _f32)

    embs = _sc_emb(*fps, emb0, emb1, emb2, emb3)
    deg = _sc_deg(dst, zrow)
    degr = deg.reshape(NC, NPAD, 1)

    pW1r = pW1.reshape(4, EMB, EMB)
    g0 = _tc_proj(embs, pW1r, pb1.reshape(1, EMB), pW2, pb2.reshape(1, EMB),
                  gW0, degr)
    acc = _sc_scatter(g0, src, dst, zrows)
    g1 = _tc_combine(acc, g0, degr, gb0.reshape(1, HID), gW1)
    acc = _sc_scatter(g1, src, dst, zrows)
    g2 = _tc_combine(acc, g1, degr, gb1.reshape(1, HID), gW2)
    acc = _sc_scatter(g2, src, dst, zrows)
    pooled = _tc_pool(acc, g2, degr, gb2.reshape(1, HID), batch_p)

    out = _tc_head(fingerprint, fpW, fpb.reshape(1, HID), pooled,
                   l1W[:HID], l1W[HID:], l1b.reshape(1, HID // 2),
                   l2W, l2b.reshape(1, 1))
    return out


# speedup vs baseline: 18.3416x; 1.9312x over previous
import functools

import jax
import jax.numpy as jnp
from jax import lax
from jax.experimental import pallas as pl
from jax.experimental.pallas import tpu as pltpu
from jax.experimental.pallas import tpu_sc as plsc

N = 10000
E = 320000
B = 64
EMB = 128
HID = 128
FP = 2048

NPAD = 10240
NC, NS = 2, 16
NW = NC * NS
EPW = E // NW
C = 80
EC = 128
ECH = 160
HH = HID // 2
RPT = NPAD // NS
DEPTH = 4
R = 256
GRID = NPAD // R

_f32 = jnp.float32


def _mesh():
    return plsc.VectorSubcoreMesh(
        core_axis_name="c", subcore_axis_name="s", num_cores=NC, num_subcores=NS
    )


def _emb_body(f0, f1, f2, f3, t0, t1, t2, t3, out, ix0, ix1, ix2, ix3, rows,
              semi, semg, sems):
    c = lax.axis_index("c")
    s = lax.axis_index("s")
    w = s * NC + c
    rbase = w * (NPAD // NW)
    fields = (f0, f1, f2, f3)
    tables = (t0, t1, t2, t3)
    ixs = (ix0, ix1, ix2, ix3)
    idr = [pltpu.async_copy(fields[f].at[pl.ds(rbase, NPAD // NW)],
                            ixs[f], semi) for f in range(4)]
    for d in idr:
        d.wait()
    gd = [None] * 16
    sd = [None] * 16

    def idx_of(t):
        f, i = t // 4, t % 4
        return f, ixs[f].at[pl.ds(i * C, C)], rbase + i * C

    for t in range(16):
        k = t % 4
        if t >= 4:
            sd[t - 4].wait()
        f, idx, _ = idx_of(t)
        gd[t] = pltpu.async_copy(tables[f].at[idx], rows.at[k], semg.at[k])
        if t >= 1:
            k1 = (t - 1) % 4
            gd[t - 1].wait()
            f1, _, ob = idx_of(t - 1)
            sd[t - 1] = pltpu.async_copy(
                rows.at[k1], out.at[f1, pl.ds(ob, C)], sems.at[k1])
    gd[15].wait()
    f1, _, ob = idx_of(15)
    sd[15] = pltpu.async_copy(rows.at[3], out.at[f1, pl.ds(ob, C)],
                              sems.at[3])
    for t in range(12, 16):
        sd[t].wait()


def _deg_body(dst3, zrow, deg_out, idxd, ones_v, acc_sh, semi, sems):
    c = lax.axis_index("c")
    s = lax.axis_index("s")
    nch = ECH // NC
    cbase = c * nch
    for i in range(EC // 16):
        ones_v[pl.ds(i * 16, 16)] = jnp.ones((16,), _f32)
    off = s * (NPAD // NS)
    di = pltpu.async_copy(dst3.at[s], idxd, semi)
    pltpu.sync_copy(zrow, acc_sh.at[pl.ds(off, NPAD // NS)])
    di.wait()
    plsc.subcore_barrier()

    def outer(j0, carry):
        for b in range(DEPTH):
            j = cbase + j0 * DEPTH + b

            @pl.when(j0 > 0)
            def _():
                pltpu.make_async_copy(ones_v, acc_sh.at[idxd.at[j]],
                                      sems.at[b]).wait()

            pltpu.async_copy(ones_v, acc_sh.at[idxd.at[j]], sems.at[b],
                             add=True)
        return carry

    lax.fori_loop(0, nch // DEPTH, outer, 0)
    for b in range(DEPTH):
        j = cbase + nch - DEPTH + b
        pltpu.make_async_copy(ones_v, acc_sh.at[idxd.at[j]], sems.at[b]).wait()
    plsc.subcore_barrier()
    pltpu.sync_copy(acc_sh.at[pl.ds(off, NPAD // NS)],
                    deg_out.at[c, pl.ds(off, NPAD // NS)])


def _scat_body(g2, src3, dst3, zrows, out, idxs, idxd, rows, acc_sh,
               semi, semg, sems):
    c = lax.axis_index("c")
    s = lax.axis_index("s")
    off = s * RPT
    gh = g2.at[c]
    cis = pltpu.async_copy(src3.at[s], idxs, semi)
    cid = pltpu.async_copy(dst3.at[s], idxd, semi)
    pltpu.sync_copy(zrows, acc_sh.at[pl.ds(off, RPT)])
    cis.wait()
    cid.wait()
    plsc.subcore_barrier()

    def outer(j0, carry):
        for b in range(DEPTH):
            j = j0 * DEPTH + b

            @pl.when(j0 > 0)
            def _():
                pltpu.make_async_copy(rows.at[b], acc_sh.at[idxd.at[j - DEPTH]],
                                      sems.at[b]).wait()

            pltpu.async_copy(gh.at[idxs.at[j]], rows.at[b], semg.at[b])
            k1 = (b - 1) % DEPTH

            if b >= 1:
                pltpu.make_async_copy(gh.at[idxs.at[j - 1]], rows.at[k1],
                                      semg.at[k1]).wait()
                pltpu.async_copy(rows.at[k1], acc_sh.at[idxd.at[j - 1]],
                                 sems.at[k1], add=True)
            else:
                @pl.when(j0 > 0)
                def _():
                    pltpu.make_async_copy(gh.at[idxs.at[j - 1]], rows.at[k1],
                                          semg.at[k1]).wait()
                    pltpu.async_copy(rows.at[k1], acc_sh.at[idxd.at[j - 1]],
                                     sems.at[k1], add=True)
        return carry

    lax.fori_loop(0, ECH // DEPTH, outer, 0)
    jl = ECH - 1
    kl = jl % DEPTH
    pltpu.make_async_copy(gh.at[idxs.at[jl]], rows.at[kl], semg.at[kl]).wait()
    pltpu.async_copy(rows.at[kl], acc_sh.at[idxd.at[jl]], sems.at[kl],
                     add=True)
    for b in range(DEPTH):
        j = ECH - DEPTH + b
        pltpu.make_async_copy(rows.at[b], acc_sh.at[idxd.at[j]],
                              sems.at[b]).wait()
    plsc.subcore_barrier()
    pltpu.sync_copy(acc_sh.at[pl.ds(off, RPT)], out.at[c, pl.ds(off, RPT)])


def _sc_emb(fp0, fp1, fp2, fp3, e0, e1, e2, e3):
    fn = pl.kernel(
        _emb_body,
        out_type=jax.ShapeDtypeStruct((4, NPAD, EMB), _f32),
        mesh=_mesh(),
        scratch_types=[
            pltpu.VMEM((NPAD // NW,), jnp.int32),
            pltpu.VMEM((NPAD // NW,), jnp.int32),
            pltpu.VMEM((NPAD // NW,), jnp.int32),
            pltpu.VMEM((NPAD // NW,), jnp.int32),
            pltpu.VMEM((4, C, EMB), _f32),
            pltpu.SemaphoreType.DMA,
            pltpu.SemaphoreType.DMA((4,)),
            pltpu.SemaphoreType.DMA((4,)),
        ],
    )
    return fn(fp0, fp1, fp2, fp3, e0, e1, e2, e3)


def _sc_deg(dst3, zrow):
    fn = pl.kernel(
        _deg_body,
        out_type=jax.ShapeDtypeStruct((NC, NPAD), _f32),
        mesh=_mesh(),
        scratch_types=[
            pltpu.VMEM((ECH, EC), jnp.int32),
            pltpu.VMEM((EC,), _f32),
            pltpu.VMEM_SHARED((NPAD,), _f32),
            pltpu.SemaphoreType.DMA,
            pltpu.SemaphoreType.DMA((DEPTH,)),
        ],
    )
    return fn(dst3, zrow)


def _sc_scatter(g2, src3, dst3, zrows):
    fn = pl.kernel(
        _scat_body,
        out_type=jax.ShapeDtypeStruct((NC, NPAD, HH), _f32),
        mesh=_mesh(),
        scratch_types=[
            pltpu.VMEM((ECH, EC), jnp.int32),
            pltpu.VMEM((ECH, EC), jnp.int32),
            pltpu.VMEM((DEPTH, EC, HH), _f32),
            pltpu.VMEM_SHARED((NPAD, HH), _f32),
            pltpu.SemaphoreType.DMA,
            pltpu.SemaphoreType.DMA((DEPTH,)),
            pltpu.SemaphoreType.DMA((DEPTH,)),
        ],
        compiler_params=pltpu.CompilerParams(use_tc_tiling_on_sc=False),
    )
    return fn(g2, src3, dst3, zrows)


def _dinv(deg_ref):
    d = deg_ref[0] + deg_ref[1] + 1.0
    return lax.rsqrt(d)


def _t0_body(embs_ref, pW1_ref, pb1_ref, pW2_ref, pb2_ref, gW0_ref, deg_ref,
             out_ref):
    h = jnp.dot(embs_ref[0], pW1_ref[0], preferred_element_type=_f32)
    for f in range(1, 4):
        h += jnp.dot(embs_ref[f], pW1_ref[f], preferred_element_type=_f32)
    h = jnp.maximum(h + pb1_ref[...], 0.0)
    h = jnp.dot(h, pW2_ref[...], preferred_element_type=_f32) + pb2_ref[...]
    u = jnp.dot(h, gW0_ref[...], preferred_element_type=_f32)
    du = _dinv(deg_ref) * u
    out_ref[0] = du[:, :HH]
    out_ref[1] = du[:, HH:]


def _tc_proj(embs, pW1r, pb1, pW2, pb2, gW0, degr):
    return pl.pallas_call(
        _t0_body,
        grid=(GRID,),
        in_specs=[
            pl.BlockSpec((4, R, EMB), lambda i: (0, i, 0)),
            pl.BlockSpec((4, EMB, EMB), lambda i: (0, 0, 0)),
            pl.BlockSpec((1, EMB), lambda i: (0, 0)),
            pl.BlockSpec((EMB, EMB), lambda i: (0, 0)),
            pl.BlockSpec((1, EMB), lambda i: (0, 0)),
            pl.BlockSpec((EMB, HID), lambda i: (0, 0)),
            pl.BlockSpec((2, R, 1), lambda i: (0, i, 0)),
        ],
        out_specs=pl.BlockSpec((2, R, HH), lambda i: (0, i, 0)),
        out_shape=jax.ShapeDtypeStruct((2, NPAD, HH), _f32),
    )(embs, pW1r, pb1, pW2, pb2, gW0, degr)


def _comb_body(acc_ref, g_ref, deg_ref, gb_ref, W_ref, out_ref):
    dinv = _dinv(deg_ref)
    m = jnp.concatenate([acc_ref[0] + g_ref[0], acc_ref[1] + g_ref[1]],
                        axis=1)
    x = jnp.maximum(dinv * m + gb_ref[...], 0.0)
    y = dinv * jnp.dot(x, W_ref[...], preferred_element_type=_f32)
    out_ref[0] = y[:, :HH]
    out_ref[1] = y[:, HH:]


def _tc_combine(acc, g, degr, gb, W):
    return pl.pallas_call(
        _comb_body,
        grid=(GRID,),
        in_specs=[
            pl.BlockSpec((2, R, HH), lambda i: (0, i, 0)),
            pl.BlockSpec((2, R, HH), lambda i: (0, i, 0)),
            pl.BlockSpec((2, R, 1), lambda i: (0, i, 0)),
            pl.BlockSpec((1, HID), lambda i: (0, 0)),
            pl.BlockSpec((HID, HID), lambda i: (0, 0)),
        ],
        out_specs=pl.BlockSpec((2, R, HH), lambda i: (0, i, 0)),
        out_shape=jax.ShapeDtypeStruct((2, NPAD, HH), _f32),
    )(acc, g, degr, gb, W)


def _pool_body(acc_ref, g_ref, deg_ref, gb_ref, batch_ref, out_ref):
    dinv = _dinv(deg_ref)
    m = jnp.concatenate([acc_ref[0] + g_ref[0], acc_ref[1] + g_ref[1]],
                        axis=1)
    x = jnp.maximum(dinv * m + gb_ref[...], 0.0)
    b = batch_ref[...]
    oh = (b == lax.broadcasted_iota(jnp.int32, (R, B), 1)).astype(_f32)
    part = lax.dot_general(oh, x, (((0,), (0,)), ((), ())),
                           preferred_element_type=_f32)

    @pl.when(pl.program_id(0) == 0)
    def _():
        out_ref[...] = part

    @pl.when(pl.program_id(0) != 0)
    def _():
        out_ref[...] += part


def _tc_pool(acc, g, degr, gb, batch_p):
    return pl.pallas_call(
        _pool_body,
        grid=(GRID,),
        in_specs=[
            pl.BlockSpec((2, R, HH), lambda i: (0, i, 0)),
            pl.BlockSpec((2, R, HH), lambda i: (0, i, 0)),
            pl.BlockSpec((2, R, 1), lambda i: (0, i, 0)),
            pl.BlockSpec((1, HID), lambda i: (0, 0)),
            pl.BlockSpec((R, 1), lambda i: (i, 0)),
        ],
        out_specs=pl.BlockSpec((B, HID), lambda i: (0, 0)),
        out_shape=jax.ShapeDtypeStruct((B, HID), _f32),
    )(acc, g, degr, gb, batch_p)


def _head_body(fp_ref, fpW_ref, fpb_ref, pooled_ref, l1a_ref, l1b_ref,
               l1bias_ref, l2W_ref, l2b_ref, out_ref):
    fpe = jnp.dot(fp_ref[...], fpW_ref[...], preferred_element_type=_f32)
    fpe = fpe + fpb_ref[...]
    z = (jnp.dot(fpe, l1a_ref[...], preferred_element_type=_f32)
         + jnp.dot(pooled_ref[...], l1b_ref[...], preferred_element_type=_f32)
         + l1bias_ref[...])
    z = jnp.maximum(z, 0.0)
    out_ref[...] = (jnp.dot(z, l2W_ref[...], preferred_element_type=_f32)
                    + l2b_ref[...])


def _tc_head(fingerprint, fpW, fpb, pooled, l1Wa, l1Wb, l1b, l2W, l2b):
    return pl.pallas_call(
        _head_body,
        out_shape=jax.ShapeDtypeStruct((B, 1), _f32),
    )(fingerprint, fpW, fpb, pooled, l1Wa, l1Wb, l1b, l2W, l2b)


def kernel(f0, f1, f2, f3, edge_index, batch, fingerprint,
           emb0, emb1, emb2, emb3, pW1, pb1, pW2, pb2,
           gW0, gb0, gW1, gb1, gW2, gb2, fpW, fpb, l1W, l1b, l2W, l2b):
    eps = E // NS
    npe = ECH * EC - eps
    pad_e = jnp.arange(npe, dtype=jnp.int32) % (NPAD - N) + N
    pad_e = jnp.broadcast_to(pad_e, (NS, npe))
    src3 = jnp.concatenate(
        [edge_index[0].reshape(NS, eps), pad_e], axis=1).reshape(NS, ECH, EC)
    dst3 = jnp.concatenate(
        [edge_index[1].reshape(NS, eps), pad_e], axis=1).reshape(NS, ECH, EC)

    pad_i = jnp.arange(NPAD - N, dtype=jnp.int32)
    fps = [jnp.concatenate([f.astype(jnp.int32), pad_i])
           for f in (f0, f1, f2, f3)]
    batch_p = jnp.concatenate(
        [batch.astype(jnp.int32), jnp.full((NPAD - N,), B, jnp.int32)]
    ).reshape(NPAD, 1)
    zrow = jnp.zeros((RPT,), _f32)
    zrows = jnp.zeros((RPT, HH), _f32)

    embs = _sc_emb(*fps, emb0, emb1, emb2, emb3)
    deg = _sc_deg(dst3, zrow)
    degr = deg.reshape(NC, NPAD, 1)

    pW1r = pW1.reshape(4, EMB, EMB)
    g0 = _tc_proj(embs, pW1r, pb1.reshape(1, EMB), pW2, pb2.reshape(1, EMB),
                  gW0, degr)
    acc = _sc_scatter(g0, src3, dst3, zrows)
    g1 = _tc_combine(acc, g0, degr, gb0.reshape(1, HID), gW1)
    acc = _sc_scatter(g1, src3, dst3, zrows)
    g2 = _tc_combine(acc, g1, degr, gb1.reshape(1, HID), gW2)
    acc = _sc_scatter(g2, src3, dst3, zrows)
    pooled = _tc_pool(acc, g2, degr, gb2.reshape(1, HID), batch_p)

    out = _tc_head(fingerprint, fpW, fpb.reshape(1, HID), pooled,
                   l1W[:HID], l1W[HID:], l1b.reshape(1, HID // 2),
                   l2W, l2b.reshape(1, 1))
    return out


# speedup vs baseline: 18.3700x; 1.0016x over previous
import functools

import jax
import jax.numpy as jnp
from jax import lax
from jax.experimental import pallas as pl
from jax.experimental.pallas import tpu as pltpu
from jax.experimental.pallas import tpu_sc as plsc

N = 10000
E = 320000
B = 64
EMB = 128
HID = 128
FP = 2048

NPAD = 10240
NC, NS = 2, 16
NW = NC * NS
EPW = E // NW
C = 80
EC = 128
ECH = 160
HH = HID // 2
RPT = NPAD // NS
DEPTH = 5
R = 256
GRID = NPAD // R

_f32 = jnp.float32


def _mesh():
    return plsc.VectorSubcoreMesh(
        core_axis_name="c", subcore_axis_name="s", num_cores=NC, num_subcores=NS
    )


def _emb_body(f0, f1, f2, f3, t0, t1, t2, t3, out, ix0, ix1, ix2, ix3, rows,
              semi, semg, sems):
    c = lax.axis_index("c")
    s = lax.axis_index("s")
    w = s * NC + c
    rbase = w * (NPAD // NW)
    fields = (f0, f1, f2, f3)
    tables = (t0, t1, t2, t3)
    ixs = (ix0, ix1, ix2, ix3)
    idr = [pltpu.async_copy(fields[f].at[pl.ds(rbase, NPAD // NW)],
                            ixs[f], semi) for f in range(4)]
    for d in idr:
        d.wait()
    gd = [None] * 16
    sd = [None] * 16

    def idx_of(t):
        f, i = t // 4, t % 4
        return f, ixs[f].at[pl.ds(i * C, C)], rbase + i * C

    for t in range(16):
        k = t % 4
        if t >= 4:
            sd[t - 4].wait()
        f, idx, _ = idx_of(t)
        gd[t] = pltpu.async_copy(tables[f].at[idx], rows.at[k], semg.at[k])
        if t >= 1:
            k1 = (t - 1) % 4
            gd[t - 1].wait()
            f1, _, ob = idx_of(t - 1)
            sd[t - 1] = pltpu.async_copy(
                rows.at[k1], out.at[f1, pl.ds(ob, C)], sems.at[k1])
    gd[15].wait()
    f1, _, ob = idx_of(15)
    sd[15] = pltpu.async_copy(rows.at[3], out.at[f1, pl.ds(ob, C)],
                              sems.at[3])
    for t in range(12, 16):
        sd[t].wait()


def _deg_body(dst3, zrow, deg_out, idxd, ones_v, acc_sh, semi, sems):
    c = lax.axis_index("c")
    s = lax.axis_index("s")
    nch = ECH // NC
    cbase = c * nch
    for i in range(EC // 16):
        ones_v[pl.ds(i * 16, 16)] = jnp.ones((16,), _f32)
    off = s * (NPAD // NS)
    di = pltpu.async_copy(dst3.at[s], idxd, semi)
    pltpu.sync_copy(zrow, acc_sh.at[pl.ds(off, NPAD // NS)])
    di.wait()
    plsc.subcore_barrier()

    def outer(j0, carry):
        for b in range(DEPTH):
            j = cbase + j0 * DEPTH + b

            @pl.when(j0 > 0)
            def _():
                pltpu.make_async_copy(ones_v, acc_sh.at[idxd.at[j]],
                                      sems.at[b]).wait()

            pltpu.async_copy(ones_v, acc_sh.at[idxd.at[j]], sems.at[b],
                             add=True)
        return carry

    lax.fori_loop(0, nch // DEPTH, outer, 0)
    for b in range(DEPTH):
        j = cbase + nch - DEPTH + b
        pltpu.make_async_copy(ones_v, acc_sh.at[idxd.at[j]], sems.at[b]).wait()
    plsc.subcore_barrier()
    pltpu.sync_copy(acc_sh.at[pl.ds(off, NPAD // NS)],
                    deg_out.at[c, pl.ds(off, NPAD // NS)])


def _scat_body(g2, src3, dst3, zrows, out, idxs, idxd, rows, acc_sh,
               semi, semg, sems):
    c = lax.axis_index("c")
    s = lax.axis_index("s")
    off = s * RPT
    gh = g2.at[c]
    cis = pltpu.async_copy(src3.at[s], idxs, semi)
    cid = pltpu.async_copy(dst3.at[s], idxd, semi)
    pltpu.sync_copy(zrows, acc_sh.at[pl.ds(off, RPT)])
    cis.wait()
    cid.wait()
    plsc.subcore_barrier()

    def outer(j0, carry):
        for b in range(DEPTH):
            j = j0 * DEPTH + b

            @pl.when(j0 > 0)
            def _():
                pltpu.make_async_copy(rows.at[b], acc_sh.at[idxd.at[j - DEPTH]],
                                      sems.at[b]).wait()

            pltpu.async_copy(gh.at[idxs.at[j]], rows.at[b], semg.at[b])
            k1 = (b - 1) % DEPTH

            if b >= 1:
                pltpu.make_async_copy(gh.at[idxs.at[j - 1]], rows.at[k1],
                                      semg.at[k1]).wait()
                pltpu.async_copy(rows.at[k1], acc_sh.at[idxd.at[j - 1]],
                                 sems.at[k1], add=True)
            else:
                @pl.when(j0 > 0)
                def _():
                    pltpu.make_async_copy(gh.at[idxs.at[j - 1]], rows.at[k1],
                                          semg.at[k1]).wait()
                    pltpu.async_copy(rows.at[k1], acc_sh.at[idxd.at[j - 1]],
                                     sems.at[k1], add=True)
        return carry

    lax.fori_loop(0, ECH // DEPTH, outer, 0)
    jl = ECH - 1
    kl = jl % DEPTH
    pltpu.make_async_copy(gh.at[idxs.at[jl]], rows.at[kl], semg.at[kl]).wait()
    pltpu.async_copy(rows.at[kl], acc_sh.at[idxd.at[jl]], sems.at[kl],
                     add=True)
    for b in range(DEPTH):
        j = ECH - DEPTH + b
        pltpu.make_async_copy(rows.at[b], acc_sh.at[idxd.at[j]],
                              sems.at[b]).wait()
    plsc.subcore_barrier()
    pltpu.sync_copy(acc_sh.at[pl.ds(off, RPT)], out.at[c, pl.ds(off, RPT)])


def _sc_emb(fp0, fp1, fp2, fp3, e0, e1, e2, e3):
    fn = pl.kernel(
        _emb_body,
        out_type=jax.ShapeDtypeStruct((4, NPAD, EMB), _f32),
        mesh=_mesh(),
        scratch_types=[
            pltpu.VMEM((NPAD // NW,), jnp.int32),
            pltpu.VMEM((NPAD // NW,), jnp.int32),
            pltpu.VMEM((NPAD // NW,), jnp.int32),
            pltpu.VMEM((NPAD // NW,), jnp.int32),
            pltpu.VMEM((4, C, EMB), _f32),
            pltpu.SemaphoreType.DMA,
            pltpu.SemaphoreType.DMA((4,)),
            pltpu.SemaphoreType.DMA((4,)),
        ],
    )
    return fn(fp0, fp1, fp2, fp3, e0, e1, e2, e3)


def _sc_deg(dst3, zrow):
    fn = pl.kernel(
        _deg_body,
        out_type=jax.ShapeDtypeStruct((NC, NPAD), _f32),
        mesh=_mesh(),
        scratch_types=[
            pltpu.VMEM((ECH, EC), jnp.int32),
            pltpu.VMEM((EC,), _f32),
            pltpu.VMEM_SHARED((NPAD,), _f32),
            pltpu.SemaphoreType.DMA,
            pltpu.SemaphoreType.DMA((DEPTH,)),
        ],
    )
    return fn(dst3, zrow)


def _sc_scatter(g2, src3, dst3, zrows):
    fn = pl.kernel(
        _scat_body,
        out_type=jax.ShapeDtypeStruct((NC, NPAD, HH), _f32),
        mesh=_mesh(),
        scratch_types=[
            pltpu.VMEM((ECH, EC), jnp.int32),
            pltpu.VMEM((ECH, EC), jnp.int32),
            pltpu.VMEM((DEPTH, EC, HH), _f32),
            pltpu.VMEM_SHARED((NPAD, HH), _f32),
            pltpu.SemaphoreType.DMA,
            pltpu.SemaphoreType.DMA((DEPTH,)),
            pltpu.SemaphoreType.DMA((DEPTH,)),
        ],
        compiler_params=pltpu.CompilerParams(use_tc_tiling_on_sc=False),
    )
    return fn(g2, src3, dst3, zrows)


def _dinv(deg_ref):
    d = deg_ref[0] + deg_ref[1] + 1.0
    return lax.rsqrt(d)


def _t0_body(embs_ref, pW1_ref, pb1_ref, pW2_ref, pb2_ref, gW0_ref, deg_ref,
             out_ref):
    h = jnp.dot(embs_ref[0], pW1_ref[0], preferred_element_type=_f32)
    for f in range(1, 4):
        h += jnp.dot(embs_ref[f], pW1_ref[f], preferred_element_type=_f32)
    h = jnp.maximum(h + pb1_ref[...], 0.0)
    h = jnp.dot(h, pW2_ref[...], preferred_element_type=_f32) + pb2_ref[...]
    u = jnp.dot(h, gW0_ref[...], preferred_element_type=_f32)
    du = _dinv(deg_ref) * u
    out_ref[0] = du[:, :HH]
    out_ref[1] = du[:, HH:]


def _tc_proj(embs, pW1r, pb1, pW2, pb2, gW0, degr):
    return pl.pallas_call(
        _t0_body,
        grid=(GRID,),
        in_specs=[
            pl.BlockSpec((4, R, EMB), lambda i: (0, i, 0)),
            pl.BlockSpec((4, EMB, EMB), lambda i: (0, 0, 0)),
            pl.BlockSpec((1, EMB), lambda i: (0, 0)),
            pl.BlockSpec((EMB, EMB), lambda i: (0, 0)),
            pl.BlockSpec((1, EMB), lambda i: (0, 0)),
            pl.BlockSpec((EMB, HID), lambda i: (0, 0)),
            pl.BlockSpec((2, R, 1), lambda i: (0, i, 0)),
        ],
        out_specs=pl.BlockSpec((2, R, HH), lambda i: (0, i, 0)),
        out_shape=jax.ShapeDtypeStruct((2, NPAD, HH), _f32),
    )(embs, pW1r, pb1, pW2, pb2, gW0, degr)


def _comb_body(acc_ref, g_ref, deg_ref, gb_ref, W_ref, out_ref):
    dinv = _dinv(deg_ref)
    m = jnp.concatenate([acc_ref[0] + g_ref[0], acc_ref[1] + g_ref[1]],
                        axis=1)
    x = jnp.maximum(dinv * m + gb_ref[...], 0.0)
    y = dinv * jnp.dot(x, W_ref[...], preferred_element_type=_f32)
    out_ref[0] = y[:, :HH]
    out_ref[1] = y[:, HH:]


def _tc_combine(acc, g, degr, gb, W):
    return pl.pallas_call(
        _comb_body,
        grid=(GRID,),
        in_specs=[
            pl.BlockSpec((2, R, HH), lambda i: (0, i, 0)),
            pl.BlockSpec((2, R, HH), lambda i: (0, i, 0)),
            pl.BlockSpec((2, R, 1), lambda i: (0, i, 0)),
            pl.BlockSpec((1, HID), lambda i: (0, 0)),
            pl.BlockSpec((HID, HID), lambda i: (0, 0)),
        ],
        out_specs=pl.BlockSpec((2, R, HH), lambda i: (0, i, 0)),
        out_shape=jax.ShapeDtypeStruct((2, NPAD, HH), _f32),
    )(acc, g, degr, gb, W)


def _pool_body(acc_ref, g_ref, deg_ref, gb_ref, batch_ref, out_ref):
    dinv = _dinv(deg_ref)
    m = jnp.concatenate([acc_ref[0] + g_ref[0], acc_ref[1] + g_ref[1]],
                        axis=1)
    x = jnp.maximum(dinv * m + gb_ref[...], 0.0)
    b = batch_ref[...]
    oh = (b == lax.broadcasted_iota(jnp.int32, (R, B), 1)).astype(_f32)
    part = lax.dot_general(oh, x, (((0,), (0,)), ((), ())),
                           preferred_element_type=_f32)

    @pl.when(pl.program_id(0) == 0)
    def _():
        out_ref[...] = part

    @pl.when(pl.program_id(0) != 0)
    def _():
        out_ref[...] += part


def _tc_pool(acc, g, degr, gb, batch_p):
    return pl.pallas_call(
        _pool_body,
        grid=(GRID,),
        in_specs=[
            pl.BlockSpec((2, R, HH), lambda i: (0, i, 0)),
            pl.BlockSpec((2, R, HH), lambda i: (0, i, 0)),
            pl.BlockSpec((2, R, 1), lambda i: (0, i, 0)),
            pl.BlockSpec((1, HID), lambda i: (0, 0)),
            pl.BlockSpec((R, 1), lambda i: (i, 0)),
        ],
        out_specs=pl.BlockSpec((B, HID), lambda i: (0, 0)),
        out_shape=jax.ShapeDtypeStruct((B, HID), _f32),
    )(acc, g, degr, gb, batch_p)


def _head_body(fp_ref, fpW_ref, fpb_ref, pooled_ref, l1a_ref, l1b_ref,
               l1bias_ref, l2W_ref, l2b_ref, out_ref):
    fpe = jnp.dot(fp_ref[...], fpW_ref[...], preferred_element_type=_f32)
    fpe = fpe + fpb_ref[...]
    z = (jnp.dot(fpe, l1a_ref[...], preferred_element_type=_f32)
         + jnp.dot(pooled_ref[...], l1b_ref[...], preferred_element_type=_f32)
         + l1bias_ref[...])
    z = jnp.maximum(z, 0.0)
    out_ref[...] = (jnp.dot(z, l2W_ref[...], preferred_element_type=_f32)
                    + l2b_ref[...])


def _tc_head(fingerprint, fpW, fpb, pooled, l1Wa, l1Wb, l1b, l2W, l2b):
    return pl.pallas_call(
        _head_body,
        out_shape=jax.ShapeDtypeStruct((B, 1), _f32),
    )(fingerprint, fpW, fpb, pooled, l1Wa, l1Wb, l1b, l2W, l2b)


def kernel(f0, f1, f2, f3, edge_index, batch, fingerprint,
           emb0, emb1, emb2, emb3, pW1, pb1, pW2, pb2,
           gW0, gb0, gW1, gb1, gW2, gb2, fpW, fpb, l1W, l1b, l2W, l2b):
    eps = E // NS
    npe = ECH * EC - eps
    pad_e = jnp.arange(npe, dtype=jnp.int32) % (NPAD - N) + N
    pad_e = jnp.broadcast_to(pad_e, (NS, npe))
    src3 = jnp.concatenate(
        [edge_index[0].reshape(NS, eps), pad_e], axis=1).reshape(NS, ECH, EC)
    dst3 = jnp.concatenate(
        [edge_index[1].reshape(NS, eps), pad_e], axis=1).reshape(NS, ECH, EC)

    pad_i = jnp.arange(NPAD - N, dtype=jnp.int32)
    fps = [jnp.concatenate([f.astype(jnp.int32), pad_i])
           for f in (f0, f1, f2, f3)]
    batch_p = jnp.concatenate(
        [batch.astype(jnp.int32), jnp.full((NPAD - N,), B, jnp.int32)]
    ).reshape(NPAD, 1)
    zrow = jnp.zeros((RPT,), _f32)
    zrows = jnp.zeros((RPT, HH), _f32)

    embs = _sc_emb(*fps, emb0, emb1, emb2, emb3)
    deg = _sc_deg(dst3, zrow)
    degr = deg.reshape(NC, NPAD, 1)

    pW1r = pW1.reshape(4, EMB, EMB)
    g0 = _tc_proj(embs, pW1r, pb1.reshape(1, EMB), pW2, pb2.reshape(1, EMB),
                  gW0, degr)
    acc = _sc_scatter(g0, src3, dst3, zrows)
    g1 = _tc_combine(acc, g0, degr, gb0.reshape(1, HID), gW1)
    acc = _sc_scatter(g1, src3, dst3, zrows)
    g2 = _tc_combine(acc, g1, degr, gb1.reshape(1, HID), gW2)
    acc = _sc_scatter(g2, src3, dst3, zrows)
    pooled = _tc_pool(acc, g2, degr, gb2.reshape(1, HID), batch_p)

    out = _tc_head(fingerprint, fpW, fpb.reshape(1, HID), pooled,
                   l1W[:HID], l1W[HID:], l1b.reshape(1, HID // 2),
                   l2W, l2b.reshape(1, 1))
    return out


# speedup vs baseline: 19.9270x; 1.0848x over previous
import functools

import jax
import jax.numpy as jnp
from jax import lax
from jax.experimental import pallas as pl
from jax.experimental.pallas import tpu as pltpu
from jax.experimental.pallas import tpu_sc as plsc

N = 10000
E = 320000
B = 64
EMB = 128
HID = 128
FP = 2048

NPAD = 10240
NC, NS = 2, 16
NW = NC * NS
EPW = E // NW
C = 80
EC = 128
ECH = 160
HH = HID // 2
RPT = NPAD // NS
DEPTH = 5
R = 512
GRID = NPAD // R

_f32 = jnp.float32


def _mesh():
    return plsc.VectorSubcoreMesh(
        core_axis_name="c", subcore_axis_name="s", num_cores=NC, num_subcores=NS
    )


def _emb_body(f0, f1, f2, f3, t0, t1, t2, t3, dst3, zrow, out, deg_out,
              ix0, ix1, ix2, ix3, rows, idxd, ones_v, dacc_sh,
              semi, semg, sems, semd, semdd):
    c = lax.axis_index("c")
    s = lax.axis_index("s")
    w = s * NC + c
    rbase = w * (NPAD // NW)
    fields = (f0, f1, f2, f3)
    tables = (t0, t1, t2, t3)
    ixs = (ix0, ix1, ix2, ix3)
    nch = ECH // NC
    cbase = c * nch
    doff = s * (NPAD // NS)
    for i in range(EC // 16):
        ones_v[pl.ds(i * 16, 16)] = jnp.ones((16,), _f32)
    di = pltpu.async_copy(dst3.at[s], idxd, semd)
    pltpu.sync_copy(zrow, dacc_sh.at[pl.ds(doff, NPAD // NS)])
    idr = [pltpu.async_copy(fields[f].at[pl.ds(rbase, NPAD // NW)],
                            ixs[f], semi) for f in range(4)]
    di.wait()
    plsc.subcore_barrier()

    def douter(j0, carry):
        for b in range(DEPTH):
            j = cbase + j0 * DEPTH + b

            @pl.when(j0 > 0)
            def _():
                pltpu.make_async_copy(ones_v, dacc_sh.at[idxd.at[j]],
                                      semdd.at[b]).wait()

            pltpu.async_copy(ones_v, dacc_sh.at[idxd.at[j]], semdd.at[b],
                             add=True)
        return carry

    lax.fori_loop(0, nch // DEPTH, douter, 0)
    for b in range(DEPTH):
        j = cbase + nch - DEPTH + b
        pltpu.make_async_copy(ones_v, dacc_sh.at[idxd.at[j]],
                              semdd.at[b]).wait()
    for d in idr:
        d.wait()
    gd = [None] * 16
    sd = [None] * 16

    def idx_of(t):
        f, i = t // 4, t % 4
        return f, ixs[f].at[pl.ds(i * C, C)], rbase + i * C

    for t in range(16):
        k = t % 4
        if t >= 4:
            sd[t - 4].wait()
        f, idx, _ = idx_of(t)
        gd[t] = pltpu.async_copy(tables[f].at[idx], rows.at[k], semg.at[k])
        if t >= 1:
            k1 = (t - 1) % 4
            gd[t - 1].wait()
            f1, _, ob = idx_of(t - 1)
            sd[t - 1] = pltpu.async_copy(
                rows.at[k1], out.at[f1, pl.ds(ob, C)], sems.at[k1])
    gd[15].wait()
    f1, _, ob = idx_of(15)
    sd[15] = pltpu.async_copy(rows.at[3], out.at[f1, pl.ds(ob, C)],
                              sems.at[3])
    for t in range(12, 16):
        sd[t].wait()
    plsc.subcore_barrier()
    pltpu.sync_copy(dacc_sh.at[pl.ds(doff, NPAD // NS)],
                    deg_out.at[c, pl.ds(doff, NPAD // NS)])


def _scat_body(g2, src3, dst3, zrows, out, idxs, idxd, rows, acc_sh,
               semi, semg, sems):
    c = lax.axis_index("c")
    s = lax.axis_index("s")
    off = s * RPT
    gh = g2.at[c]
    cis = pltpu.async_copy(src3.at[s], idxs, semi)
    cid = pltpu.async_copy(dst3.at[s], idxd, semi)
    pltpu.sync_copy(zrows, acc_sh.at[pl.ds(off, RPT)])
    cis.wait()
    cid.wait()
    plsc.subcore_barrier()

    def outer(j0, carry):
        for b in range(DEPTH):
            j = j0 * DEPTH + b

            @pl.when(j0 > 0)
            def _():
                pltpu.make_async_copy(rows.at[b], acc_sh.at[idxd.at[j - DEPTH]],
                                      sems.at[b]).wait()

            pltpu.async_copy(gh.at[idxs.at[j]], rows.at[b], semg.at[b])
            k1 = (b - 1) % DEPTH

            if b >= 1:
                pltpu.make_async_copy(gh.at[idxs.at[j - 1]], rows.at[k1],
                                      semg.at[k1]).wait()
                pltpu.async_copy(rows.at[k1], acc_sh.at[idxd.at[j - 1]],
                                 sems.at[k1], add=True)
            else:
                @pl.when(j0 > 0)
                def _():
                    pltpu.make_async_copy(gh.at[idxs.at[j - 1]], rows.at[k1],
                                          semg.at[k1]).wait()
                    pltpu.async_copy(rows.at[k1], acc_sh.at[idxd.at[j - 1]],
                                     sems.at[k1], add=True)
        return carry

    lax.fori_loop(0, ECH // DEPTH, outer, 0)
    jl = ECH - 1
    kl = jl % DEPTH
    pltpu.make_async_copy(gh.at[idxs.at[jl]], rows.at[kl], semg.at[kl]).wait()
    pltpu.async_copy(rows.at[kl], acc_sh.at[idxd.at[jl]], sems.at[kl],
                     add=True)
    for b in range(DEPTH):
        j = ECH - DEPTH + b
        pltpu.make_async_copy(rows.at[b], acc_sh.at[idxd.at[j]],
                              sems.at[b]).wait()
    plsc.subcore_barrier()
    pltpu.sync_copy(acc_sh.at[pl.ds(off, RPT)], out.at[c, pl.ds(off, RPT)])


def _sc_emb(fp0, fp1, fp2, fp3, e0, e1, e2, e3, dst3, zrow):
    fn = pl.kernel(
        _emb_body,
        out_type=(jax.ShapeDtypeStruct((4, NPAD, EMB), _f32),
                  jax.ShapeDtypeStruct((NC, NPAD), _f32)),
        mesh=_mesh(),
        scratch_types=[
            pltpu.VMEM((NPAD // NW,), jnp.int32),
            pltpu.VMEM((NPAD // NW,), jnp.int32),
            pltpu.VMEM((NPAD // NW,), jnp.int32),
            pltpu.VMEM((NPAD // NW,), jnp.int32),
            pltpu.VMEM((4, C, EMB), _f32),
            pltpu.VMEM((ECH, EC), jnp.int32),
            pltpu.VMEM((EC,), _f32),
            pltpu.VMEM_SHARED((NPAD,), _f32),
            pltpu.SemaphoreType.DMA,
            pltpu.SemaphoreType.DMA((4,)),
            pltpu.SemaphoreType.DMA((4,)),
            pltpu.SemaphoreType.DMA,
            pltpu.SemaphoreType.DMA((DEPTH,)),
        ],
    )
    return fn(fp0, fp1, fp2, fp3, e0, e1, e2, e3, dst3, zrow)


def _sc_scatter(g2, src3, dst3, zrows):
    fn = pl.kernel(
        _scat_body,
        out_type=jax.ShapeDtypeStruct((NC, NPAD, HH), _f32),
        mesh=_mesh(),
        scratch_types=[
            pltpu.VMEM((ECH, EC), jnp.int32),
            pltpu.VMEM((ECH, EC), jnp.int32),
            pltpu.VMEM((DEPTH, EC, HH), _f32),
            pltpu.VMEM_SHARED((NPAD, HH), _f32),
            pltpu.SemaphoreType.DMA,
            pltpu.SemaphoreType.DMA((DEPTH,)),
            pltpu.SemaphoreType.DMA((DEPTH,)),
        ],
        compiler_params=pltpu.CompilerParams(use_tc_tiling_on_sc=False),
    )
    return fn(g2, src3, dst3, zrows)


def _dinv(deg_ref):
    d = deg_ref[0] + deg_ref[1] + 1.0
    return lax.rsqrt(d)


def _t0_body(embs_ref, pW1_ref, pb1_ref, pW2_ref, pb2_ref, gW0_ref, deg_ref,
             out_ref):
    h = jnp.dot(embs_ref[0], pW1_ref[0], preferred_element_type=_f32)
    for f in range(1, 4):
        h += jnp.dot(embs_ref[f], pW1_ref[f], preferred_element_type=_f32)
    h = jnp.maximum(h + pb1_ref[...], 0.0)
    h = jnp.dot(h, pW2_ref[...], preferred_element_type=_f32) + pb2_ref[...]
    dinv = _dinv(deg_ref)
    out_ref[0] = dinv * jnp.dot(h, gW0_ref[:, :HH],
                                preferred_element_type=_f32)
    out_ref[1] = dinv * jnp.dot(h, gW0_ref[:, HH:],
                                preferred_element_type=_f32)


def _tc_proj(embs, pW1r, pb1, pW2, pb2, gW0, degr):
    return pl.pallas_call(
        _t0_body,
        grid=(GRID,),
        in_specs=[
            pl.BlockSpec((4, R, EMB), lambda i: (0, i, 0)),
            pl.BlockSpec((4, EMB, EMB), lambda i: (0, 0, 0)),
            pl.BlockSpec((1, EMB), lambda i: (0, 0)),
            pl.BlockSpec((EMB, EMB), lambda i: (0, 0)),
            pl.BlockSpec((1, EMB), lambda i: (0, 0)),
            pl.BlockSpec((EMB, HID), lambda i: (0, 0)),
            pl.BlockSpec((2, R, 1), lambda i: (0, i, 0)),
        ],
        out_specs=pl.BlockSpec((2, R, HH), lambda i: (0, i, 0)),
        out_shape=jax.ShapeDtypeStruct((2, NPAD, HH), _f32),
    )(embs, pW1r, pb1, pW2, pb2, gW0, degr)


def _halves(acc_ref, g_ref, deg_ref, gb_ref):
    dinv = _dinv(deg_ref)
    xlo = jnp.maximum(dinv * (acc_ref[0] + g_ref[0]) + gb_ref[0], 0.0)
    xhi = jnp.maximum(dinv * (acc_ref[1] + g_ref[1]) + gb_ref[1], 0.0)
    return dinv, xlo, xhi


def _comb_body(acc_ref, g_ref, deg_ref, gb_ref, W_ref, out_ref):
    dinv, xlo, xhi = _halves(acc_ref, g_ref, deg_ref, gb_ref)
    W = W_ref
    ylo = (jnp.dot(xlo, W[:HH, :HH], preferred_element_type=_f32)
           + jnp.dot(xhi, W[HH:, :HH], preferred_element_type=_f32))
    yhi = (jnp.dot(xlo, W[:HH, HH:], preferred_element_type=_f32)
           + jnp.dot(xhi, W[HH:, HH:], preferred_element_type=_f32))
    out_ref[0] = dinv * ylo
    out_ref[1] = dinv * yhi


def _tc_combine(acc, g, degr, gb2, W):
    return pl.pallas_call(
        _comb_body,
        grid=(GRID,),
        in_specs=[
            pl.BlockSpec((2, R, HH), lambda i: (0, i, 0)),
            pl.BlockSpec((2, R, HH), lambda i: (0, i, 0)),
            pl.BlockSpec((2, R, 1), lambda i: (0, i, 0)),
            pl.BlockSpec((2, 1, HH), lambda i: (0, 0, 0)),
            pl.BlockSpec((HID, HID), lambda i: (0, 0)),
        ],
        out_specs=pl.BlockSpec((2, R, HH), lambda i: (0, i, 0)),
        out_shape=jax.ShapeDtypeStruct((2, NPAD, HH), _f32),
    )(acc, g, degr, gb2, W)


def _pool_head_body(acc_ref, g_ref, deg_ref, gb_ref, batch_ref, fp_ref,
                    fpW_ref, fpb_ref, l1a_ref, l1b_ref, l1bias_ref,
                    l2W_ref, l2b_ref, out_ref, pacc):
    i = pl.program_id(0)
    _, xlo, xhi = _halves(acc_ref, g_ref, deg_ref, gb_ref)
    bt = batch_ref[...]
    oh = (bt == lax.broadcasted_iota(jnp.int32, (R, B), 1)).astype(_f32)
    cd = (((0,), (0,)), ((), ()))
    plo = lax.dot_general(oh, xlo, cd, preferred_element_type=_f32)
    phi = lax.dot_general(oh, xhi, cd, preferred_element_type=_f32)

    @pl.when(i == 0)
    def _():
        pacc[0] = plo
        pacc[1] = phi

    @pl.when(i != 0)
    def _():
        pacc[0] += plo
        pacc[1] += phi

    @pl.when(i == GRID - 1)
    def _():
        fpe = jnp.dot(fp_ref[...], fpW_ref[...], preferred_element_type=_f32)
        fpe = fpe + fpb_ref[...]
        z = (jnp.dot(fpe, l1a_ref[...], preferred_element_type=_f32)
             + jnp.dot(pacc[0], l1b_ref[:HH], preferred_element_type=_f32)
             + jnp.dot(pacc[1], l1b_ref[HH:], preferred_element_type=_f32)
             + l1bias_ref[...])
        z = jnp.maximum(z, 0.0)
        out_ref[...] = (jnp.dot(z, l2W_ref[...], preferred_element_type=_f32)
                        + l2b_ref[...])


def _tc_pool_head(acc, g, degr, gb2, batch_p, fingerprint, fpW, fpb,
                  l1Wa, l1Wb, l1b, l2W, l2b):
    return pl.pallas_call(
        _pool_head_body,
        grid=(GRID,),
        in_specs=[
            pl.BlockSpec((2, R, HH), lambda i: (0, i, 0)),
            pl.BlockSpec((2, R, HH), lambda i: (0, i, 0)),
            pl.BlockSpec((2, R, 1), lambda i: (0, i, 0)),
            pl.BlockSpec((2, 1, HH), lambda i: (0, 0, 0)),
            pl.BlockSpec((R, 1), lambda i: (i, 0)),
            pl.BlockSpec((B, FP), lambda i: (0, 0)),
            pl.BlockSpec((FP, HID), lambda i: (0, 0)),
            pl.BlockSpec((1, HID), lambda i: (0, 0)),
            pl.BlockSpec((HID, HID // 2), lambda i: (0, 0)),
            pl.BlockSpec((HID, HID // 2), lambda i: (0, 0)),
            pl.BlockSpec((1, HID // 2), lambda i: (0, 0)),
            pl.BlockSpec((HID // 2, 1), lambda i: (0, 0)),
            pl.BlockSpec((1, 1), lambda i: (0, 0)),
        ],
        out_specs=pl.BlockSpec((B, 1), lambda i: (0, 0)),
        out_shape=jax.ShapeDtypeStruct((B, 1), _f32),
        scratch_shapes=[pltpu.VMEM((2, B, HH), _f32)],
    )(acc, g, degr, gb2, batch_p, fingerprint, fpW, fpb,
      l1Wa, l1Wb, l1b, l2W, l2b)


def kernel(f0, f1, f2, f3, edge_index, batch, fingerprint,
           emb0, emb1, emb2, emb3, pW1, pb1, pW2, pb2,
           gW0, gb0, gW1, gb1, gW2, gb2, fpW, fpb, l1W, l1b, l2W, l2b):
    eps = E // NS
    npe = ECH * EC - eps
    pad_e = jnp.arange(npe, dtype=jnp.int32) % (NPAD - N) + N
    pad_e = jnp.broadcast_to(pad_e, (NS, npe))
    src3 = jnp.concatenate(
        [edge_index[0].reshape(NS, eps), pad_e], axis=1).reshape(NS, ECH, EC)
    dst3 = jnp.concatenate(
        [edge_index[1].reshape(NS, eps), pad_e], axis=1).reshape(NS, ECH, EC)

    pad_i = jnp.arange(NPAD - N, dtype=jnp.int32)
    fps = [jnp.concatenate([f.astype(jnp.int32), pad_i])
           for f in (f0, f1, f2, f3)]
    batch_p = jnp.concatenate(
        [batch.astype(jnp.int32), jnp.full((NPAD - N,), B, jnp.int32)]
    ).reshape(NPAD, 1)
    zrow = jnp.zeros((RPT,), _f32)
    zrows = jnp.zeros((RPT, HH), _f32)

    embs, deg = _sc_emb(*fps, emb0, emb1, emb2, emb3, dst3, zrow)
    degr = deg.reshape(NC, NPAD, 1)

    pW1r = pW1.reshape(4, EMB, EMB)
    g0 = _tc_proj(embs, pW1r, pb1.reshape(1, EMB), pW2, pb2.reshape(1, EMB),
                  gW0, degr)
    acc = _sc_scatter(g0, src3, dst3, zrows)
    g1 = _tc_combine(acc, g0, degr, gb0.reshape(2, 1, HH), gW1)
    acc = _sc_scatter(g1, src3, dst3, zrows)
    g2 = _tc_combine(acc, g1, degr, gb1.reshape(2, 1, HH), gW2)
    acc = _sc_scatter(g2, src3, dst3, zrows)
    out = _tc_pool_head(acc, g2, degr, gb2.reshape(2, 1, HH), batch_p,
                        fingerprint, fpW, fpb.reshape(1, HID),
                        l1W[:HID], l1W[HID:], l1b.reshape(1, HID // 2),
                        l2W, l2b.reshape(1, 1))
    return out


# speedup vs baseline: 20.8712x; 1.0474x over previous
import functools

import jax
import jax.numpy as jnp
from jax import lax
from jax.experimental import pallas as pl
from jax.experimental.pallas import tpu as pltpu
from jax.experimental.pallas import tpu_sc as plsc

N = 10000
E = 320000
B = 64
EMB = 128
HID = 128
FP = 2048

NPAD = 10240
NC, NS = 2, 16
NW = NC * NS
EPW = E // NW
C = 80
EC = 128
ECH = 160
HH = HID // 2
RPT = NPAD // NS
DEPTH = 5
R = 1024
GRID = NPAD // R

_f32 = jnp.float32


def _mesh():
    return plsc.VectorSubcoreMesh(
        core_axis_name="c", subcore_axis_name="s", num_cores=NC, num_subcores=NS
    )


def _emb_body(f0, f1, f2, f3, t0, t1, t2, t3, dst3, zrow, out, deg_out,
              ix0, ix1, ix2, ix3, rows, idxd, ones_v, dacc_sh,
              semi, semg, sems, semd, semdd):
    c = lax.axis_index("c")
    s = lax.axis_index("s")
    w = s * NC + c
    rbase = w * (NPAD // NW)
    fields = (f0, f1, f2, f3)
    tables = (t0, t1, t2, t3)
    ixs = (ix0, ix1, ix2, ix3)
    nch = ECH // NC
    cbase = c * nch
    doff = s * (NPAD // NS)
    for i in range(EC // 16):
        ones_v[pl.ds(i * 16, 16)] = jnp.ones((16,), _f32)
    di = pltpu.async_copy(dst3.at[s], idxd, semd)
    pltpu.sync_copy(zrow, dacc_sh.at[pl.ds(doff, NPAD // NS)])
    idr = [pltpu.async_copy(fields[f].at[pl.ds(rbase, NPAD // NW)],
                            ixs[f], semi) for f in range(4)]
    di.wait()
    plsc.subcore_barrier()

    def douter(j0, carry):
        for b in range(DEPTH):
            j = cbase + j0 * DEPTH + b

            @pl.when(j0 > 0)
            def _():
                pltpu.make_async_copy(ones_v, dacc_sh.at[idxd.at[j]],
                                      semdd.at[b]).wait()

            pltpu.async_copy(ones_v, dacc_sh.at[idxd.at[j]], semdd.at[b],
                             add=True)
        return carry

    lax.fori_loop(0, nch // DEPTH, douter, 0)
    for b in range(DEPTH):
        j = cbase + nch - DEPTH + b
        pltpu.make_async_copy(ones_v, dacc_sh.at[idxd.at[j]],
                              semdd.at[b]).wait()
    for d in idr:
        d.wait()
    gd = [None] * 16
    sd = [None] * 16

    def idx_of(t):
        f, i = t // 4, t % 4
        return f, ixs[f].at[pl.ds(i * C, C)], rbase + i * C

    for t in range(16):
        k = t % 4
        if t >= 4:
            sd[t - 4].wait()
        f, idx, _ = idx_of(t)
        gd[t] = pltpu.async_copy(tables[f].at[idx], rows.at[k], semg.at[k])
        if t >= 1:
            k1 = (t - 1) % 4
            gd[t - 1].wait()
            f1, _, ob = idx_of(t - 1)
            sd[t - 1] = pltpu.async_copy(
                rows.at[k1], out.at[f1, pl.ds(ob, C)], sems.at[k1])
    gd[15].wait()
    f1, _, ob = idx_of(15)
    sd[15] = pltpu.async_copy(rows.at[3], out.at[f1, pl.ds(ob, C)],
                              sems.at[3])
    for t in range(12, 16):
        sd[t].wait()
    plsc.subcore_barrier()
    pltpu.sync_copy(dacc_sh.at[pl.ds(doff, NPAD // NS)],
                    deg_out.at[c, pl.ds(doff, NPAD // NS)])


def _scat_body(g2, src3, dst3, zrows, out, idxs, idxd, rows, acc_sh,
               semi, semg, sems):
    c = lax.axis_index("c")
    s = lax.axis_index("s")
    off = s * RPT
    gh = g2.at[c]
    cis = pltpu.async_copy(src3.at[s], idxs, semi)
    cid = pltpu.async_copy(dst3.at[s], idxd, semi)
    pltpu.sync_copy(zrows, acc_sh.at[pl.ds(off, RPT)])
    cis.wait()
    cid.wait()
    plsc.subcore_barrier()

    def outer(j0, carry):
        for b in range(DEPTH):
            j = j0 * DEPTH + b

            @pl.when(j0 > 0)
            def _():
                pltpu.make_async_copy(rows.at[b], acc_sh.at[idxd.at[j - DEPTH]],
                                      sems.at[b]).wait()

            pltpu.async_copy(gh.at[idxs.at[j]], rows.at[b], semg.at[b])
            k1 = (b - 1) % DEPTH

            if b >= 1:
                pltpu.make_async_copy(gh.at[idxs.at[j - 1]], rows.at[k1],
                                      semg.at[k1]).wait()
                pltpu.async_copy(rows.at[k1], acc_sh.at[idxd.at[j - 1]],
                                 sems.at[k1], add=True)
            else:
                @pl.when(j0 > 0)
                def _():
                    pltpu.make_async_copy(gh.at[idxs.at[j - 1]], rows.at[k1],
                                          semg.at[k1]).wait()
                    pltpu.async_copy(rows.at[k1], acc_sh.at[idxd.at[j - 1]],
                                     sems.at[k1], add=True)
        return carry

    lax.fori_loop(0, ECH // DEPTH, outer, 0)
    jl = ECH - 1
    kl = jl % DEPTH
    pltpu.make_async_copy(gh.at[idxs.at[jl]], rows.at[kl], semg.at[kl]).wait()
    pltpu.async_copy(rows.at[kl], acc_sh.at[idxd.at[jl]], sems.at[kl],
                     add=True)
    for b in range(DEPTH):
        j = ECH - DEPTH + b
        pltpu.make_async_copy(rows.at[b], acc_sh.at[idxd.at[j]],
                              sems.at[b]).wait()
    plsc.subcore_barrier()
    pltpu.sync_copy(acc_sh.at[pl.ds(off, RPT)], out.at[c, pl.ds(off, RPT)])


def _sc_emb(fp0, fp1, fp2, fp3, e0, e1, e2, e3, dst3, zrow):
    fn = pl.kernel(
        _emb_body,
        out_type=(jax.ShapeDtypeStruct((4, NPAD, EMB), _f32),
                  jax.ShapeDtypeStruct((NC, NPAD), _f32)),
        mesh=_mesh(),
        scratch_types=[
            pltpu.VMEM((NPAD // NW,), jnp.int32),
            pltpu.VMEM((NPAD // NW,), jnp.int32),
            pltpu.VMEM((NPAD // NW,), jnp.int32),
            pltpu.VMEM((NPAD // NW,), jnp.int32),
            pltpu.VMEM((4, C, EMB), _f32),
            pltpu.VMEM((ECH, EC), jnp.int32),
            pltpu.VMEM((EC,), _f32),
            pltpu.VMEM_SHARED((NPAD,), _f32),
            pltpu.SemaphoreType.DMA,
            pltpu.SemaphoreType.DMA((4,)),
            pltpu.SemaphoreType.DMA((4,)),
            pltpu.SemaphoreType.DMA,
            pltpu.SemaphoreType.DMA((DEPTH,)),
        ],
    )
    return fn(fp0, fp1, fp2, fp3, e0, e1, e2, e3, dst3, zrow)


def _sc_scatter(g2, src3, dst3, zrows):
    fn = pl.kernel(
        _scat_body,
        out_type=jax.ShapeDtypeStruct((NC, NPAD, HH), _f32),
        mesh=_mesh(),
        scratch_types=[
            pltpu.VMEM((ECH, EC), jnp.int32),
            pltpu.VMEM((ECH, EC), jnp.int32),
            pltpu.VMEM((DEPTH, EC, HH), _f32),
            pltpu.VMEM_SHARED((NPAD, HH), _f32),
            pltpu.SemaphoreType.DMA,
            pltpu.SemaphoreType.DMA((DEPTH,)),
            pltpu.SemaphoreType.DMA((DEPTH,)),
        ],
        compiler_params=pltpu.CompilerParams(use_tc_tiling_on_sc=False),
    )
    return fn(g2, src3, dst3, zrows)


def _dinv(deg_ref):
    d = deg_ref[0] + deg_ref[1] + 1.0
    return lax.rsqrt(d)


def _t0_body(embs_ref, pW1_ref, pb1_ref, pW2_ref, pb2_ref, gW0_ref, deg_ref,
             out_ref):
    h = jnp.dot(embs_ref[0], pW1_ref[0], preferred_element_type=_f32)
    for f in range(1, 4):
        h += jnp.dot(embs_ref[f], pW1_ref[f], preferred_element_type=_f32)
    h = jnp.maximum(h + pb1_ref[...], 0.0)
    h = jnp.dot(h, pW2_ref[...], preferred_element_type=_f32) + pb2_ref[...]
    dinv = _dinv(deg_ref)
    out_ref[0] = dinv * jnp.dot(h, gW0_ref[:, :HH],
                                preferred_element_type=_f32)
    out_ref[1] = dinv * jnp.dot(h, gW0_ref[:, HH:],
                                preferred_element_type=_f32)


def _tc_proj(embs, pW1r, pb1, pW2, pb2, gW0, degr):
    return pl.pallas_call(
        _t0_body,
        grid=(GRID,),
        in_specs=[
            pl.BlockSpec((4, R, EMB), lambda i: (0, i, 0)),
            pl.BlockSpec((4, EMB, EMB), lambda i: (0, 0, 0)),
            pl.BlockSpec((1, EMB), lambda i: (0, 0)),
            pl.BlockSpec((EMB, EMB), lambda i: (0, 0)),
            pl.BlockSpec((1, EMB), lambda i: (0, 0)),
            pl.BlockSpec((EMB, HID), lambda i: (0, 0)),
            pl.BlockSpec((2, R, 1), lambda i: (0, i, 0)),
        ],
        out_specs=pl.BlockSpec((2, R, HH), lambda i: (0, i, 0)),
        out_shape=jax.ShapeDtypeStruct((2, NPAD, HH), _f32),
    )(embs, pW1r, pb1, pW2, pb2, gW0, degr)


def _halves(acc_ref, g_ref, deg_ref, gb_ref):
    dinv = _dinv(deg_ref)
    xlo = jnp.maximum(dinv * (acc_ref[0] + g_ref[0]) + gb_ref[0], 0.0)
    xhi = jnp.maximum(dinv * (acc_ref[1] + g_ref[1]) + gb_ref[1], 0.0)
    return dinv, xlo, xhi


def _comb_body(acc_ref, g_ref, deg_ref, gb_ref, W_ref, out_ref):
    dinv, xlo, xhi = _halves(acc_ref, g_ref, deg_ref, gb_ref)
    W = W_ref
    ylo = (jnp.dot(xlo, W[:HH, :HH], preferred_element_type=_f32)
           + jnp.dot(xhi, W[HH:, :HH], preferred_element_type=_f32))
    yhi = (jnp.dot(xlo, W[:HH, HH:], preferred_element_type=_f32)
           + jnp.dot(xhi, W[HH:, HH:], preferred_element_type=_f32))
    out_ref[0] = dinv * ylo
    out_ref[1] = dinv * yhi


def _tc_combine(acc, g, degr, gb2, W):
    return pl.pallas_call(
        _comb_body,
        grid=(GRID,),
        in_specs=[
            pl.BlockSpec((2, R, HH), lambda i: (0, i, 0)),
            pl.BlockSpec((2, R, HH), lambda i: (0, i, 0)),
            pl.BlockSpec((2, R, 1), lambda i: (0, i, 0)),
            pl.BlockSpec((2, 1, HH), lambda i: (0, 0, 0)),
            pl.BlockSpec((HID, HID), lambda i: (0, 0)),
        ],
        out_specs=pl.BlockSpec((2, R, HH), lambda i: (0, i, 0)),
        out_shape=jax.ShapeDtypeStruct((2, NPAD, HH), _f32),
    )(acc, g, degr, gb2, W)


def _pool_head_body(acc_ref, g_ref, deg_ref, gb_ref, batch_ref, fp_ref,
                    fpW_ref, fpb_ref, l1a_ref, l1b_ref, l1bias_ref,
                    l2W_ref, l2b_ref, out_ref, pacc):
    i = pl.program_id(0)
    _, xlo, xhi = _halves(acc_ref, g_ref, deg_ref, gb_ref)
    bt = batch_ref[...]
    oh = (bt == lax.broadcasted_iota(jnp.int32, (R, B), 1)).astype(_f32)
    cd = (((0,), (0,)), ((), ()))
    plo = lax.dot_general(oh, xlo, cd, preferred_element_type=_f32)
    phi = lax.dot_general(oh, xhi, cd, preferred_element_type=_f32)

    @pl.when(i == 0)
    def _():
        pacc[0] = plo
        pacc[1] = phi

    @pl.when(i != 0)
    def _():
        pacc[0] += plo
        pacc[1] += phi

    @pl.when(i == GRID - 1)
    def _():
        fpe = jnp.dot(fp_ref[...], fpW_ref[...], preferred_element_type=_f32)
        fpe = fpe + fpb_ref[...]
        z = (jnp.dot(fpe, l1a_ref[...], preferred_element_type=_f32)
             + jnp.dot(pacc[0], l1b_ref[:HH], preferred_element_type=_f32)
             + jnp.dot(pacc[1], l1b_ref[HH:], preferred_element_type=_f32)
             + l1bias_ref[...])
        z = jnp.maximum(z, 0.0)
        out_ref[...] = (jnp.dot(z, l2W_ref[...], preferred_element_type=_f32)
                        + l2b_ref[...])


def _tc_pool_head(acc, g, degr, gb2, batch_p, fingerprint, fpW, fpb,
                  l1Wa, l1Wb, l1b, l2W, l2b):
    return pl.pallas_call(
        _pool_head_body,
        grid=(GRID,),
        in_specs=[
            pl.BlockSpec((2, R, HH), lambda i: (0, i, 0)),
            pl.BlockSpec((2, R, HH), lambda i: (0, i, 0)),
            pl.BlockSpec((2, R, 1), lambda i: (0, i, 0)),
            pl.BlockSpec((2, 1, HH), lambda i: (0, 0, 0)),
            pl.BlockSpec((R, 1), lambda i: (i, 0)),
            pl.BlockSpec((B, FP), lambda i: (0, 0)),
            pl.BlockSpec((FP, HID), lambda i: (0, 0)),
            pl.BlockSpec((1, HID), lambda i: (0, 0)),
            pl.BlockSpec((HID, HID // 2), lambda i: (0, 0)),
            pl.BlockSpec((HID, HID // 2), lambda i: (0, 0)),
            pl.BlockSpec((1, HID // 2), lambda i: (0, 0)),
            pl.BlockSpec((HID // 2, 1), lambda i: (0, 0)),
            pl.BlockSpec((1, 1), lambda i: (0, 0)),
        ],
        out_specs=pl.BlockSpec((B, 1), lambda i: (0, 0)),
        out_shape=jax.ShapeDtypeStruct((B, 1), _f32),
        scratch_shapes=[pltpu.VMEM((2, B, HH), _f32)],
    )(acc, g, degr, gb2, batch_p, fingerprint, fpW, fpb,
      l1Wa, l1Wb, l1b, l2W, l2b)


def kernel(f0, f1, f2, f3, edge_index, batch, fingerprint,
           emb0, emb1, emb2, emb3, pW1, pb1, pW2, pb2,
           gW0, gb0, gW1, gb1, gW2, gb2, fpW, fpb, l1W, l1b, l2W, l2b):
    eps = E // NS
    npe = ECH * EC - eps
    pad_e = jnp.arange(npe, dtype=jnp.int32) % (NPAD - N) + N
    pad_e = jnp.broadcast_to(pad_e, (NS, npe))
    src3 = jnp.concatenate(
        [edge_index[0].reshape(NS, eps), pad_e], axis=1).reshape(NS, ECH, EC)
    dst3 = jnp.concatenate(
        [edge_index[1].reshape(NS, eps), pad_e], axis=1).reshape(NS, ECH, EC)

    pad_i = jnp.arange(NPAD - N, dtype=jnp.int32)
    fps = [jnp.concatenate([f.astype(jnp.int32), pad_i])
           for f in (f0, f1, f2, f3)]
    batch_p = jnp.concatenate(
        [batch.astype(jnp.int32), jnp.full((NPAD - N,), B, jnp.int32)]
    ).reshape(NPAD, 1)
    zrow = jnp.zeros((RPT,), _f32)
    zrows = jnp.zeros((RPT, HH), _f32)

    embs, deg = _sc_emb(*fps, emb0, emb1, emb2, emb3, dst3, zrow)
    degr = deg.reshape(NC, NPAD, 1)

    pW1r = pW1.reshape(4, EMB, EMB)
    g0 = _tc_proj(embs, pW1r, pb1.reshape(1, EMB), pW2, pb2.reshape(1, EMB),
                  gW0, degr)
    acc = _sc_scatter(g0, src3, dst3, zrows)
    g1 = _tc_combine(acc, g0, degr, gb0.reshape(2, 1, HH), gW1)
    acc = _sc_scatter(g1, src3, dst3, zrows)
    g2 = _tc_combine(acc, g1, degr, gb1.reshape(2, 1, HH), gW2)
    acc = _sc_scatter(g2, src3, dst3, zrows)
    out = _tc_pool_head(acc, g2, degr, gb2.reshape(2, 1, HH), batch_p,
                        fingerprint, fpW, fpb.reshape(1, HID),
                        l1W[:HID], l1W[HID:], l1b.reshape(1, HID // 2),
                        l2W, l2b.reshape(1, 1))
    return out


# speedup vs baseline: 24.7110x; 1.1840x over previous
import functools

import jax
import jax.numpy as jnp
from jax import lax
from jax.experimental import pallas as pl
from jax.experimental.pallas import tpu as pltpu
from jax.experimental.pallas import tpu_sc as plsc

N = 10000
E = 320000
B = 64
EMB = 128
HID = 128
FP = 2048

NPAD = 10240
NC, NS = 2, 16
NW = NC * NS
EPW = E // NW
C = 80
EC = 128
NCHW = 80
SPH = NCHW // 2
HH = HID // 2
RPT = NPAD // NS
DEPTH = 5
DEPTH2 = 2
R = 1024
GRID = NPAD // R

_f32 = jnp.float32


def _mesh():
    return plsc.VectorSubcoreMesh(
        core_axis_name="c", subcore_axis_name="s", num_cores=NC, num_subcores=NS
    )


def _emb_body(f0, f1, f2, f3, t0, t1, t2, t3, dst3, zrow, out, deg_out,
              ix0, ix1, ix2, ix3, rows, idxd, ones_v, dacc_sh,
              semi, semg, sems, semd, semdd):
    c = lax.axis_index("c")
    s = lax.axis_index("s")
    w = s * NC + c
    rbase = w * (NPAD // NW)
    fields = (f0, f1, f2, f3)
    tables = (t0, t1, t2, t3)
    ixs = (ix0, ix1, ix2, ix3)
    doff = s * (NPAD // NS)
    for i in range(EC // 16):
        ones_v[pl.ds(i * 16, 16)] = jnp.ones((16,), _f32)
    di = pltpu.async_copy(dst3.at[w], idxd, semd)
    pltpu.sync_copy(zrow, dacc_sh.at[pl.ds(doff, NPAD // NS)])
    idr = [pltpu.async_copy(fields[f].at[pl.ds(rbase, NPAD // NW)],
                            ixs[f], semi) for f in range(4)]
    di.wait()
    plsc.subcore_barrier()

    def douter(j0, carry):
        for b in range(DEPTH):
            j = j0 * DEPTH + b

            @pl.when(j0 > 0)
            def _():
                pltpu.make_async_copy(ones_v, dacc_sh.at[idxd.at[j]],
                                      semdd.at[b]).wait()

            pltpu.async_copy(ones_v, dacc_sh.at[idxd.at[j]], semdd.at[b],
                             add=True)
        return carry

    lax.fori_loop(0, NCHW // DEPTH, douter, 0)
    for b in range(DEPTH):
        j = NCHW - DEPTH + b
        pltpu.make_async_copy(ones_v, dacc_sh.at[idxd.at[j]],
                              semdd.at[b]).wait()
    for d in idr:
        d.wait()
    gd = [None] * 16
    sd = [None] * 16

    def idx_of(t):
        f, i = t // 4, t % 4
        return f, ixs[f].at[pl.ds(i * C, C)], rbase + i * C

    for t in range(16):
        k = t % 4
        if t >= 4:
            sd[t - 4].wait()
        f, idx, _ = idx_of(t)
        gd[t] = pltpu.async_copy(tables[f].at[idx], rows.at[k], semg.at[k])
        if t >= 1:
            k1 = (t - 1) % 4
            gd[t - 1].wait()
            f1, _, ob = idx_of(t - 1)
            sd[t - 1] = pltpu.async_copy(
                rows.at[k1], out.at[f1, pl.ds(ob, C)], sems.at[k1])
    gd[15].wait()
    f1, _, ob = idx_of(15)
    sd[15] = pltpu.async_copy(rows.at[3], out.at[f1, pl.ds(ob, C)],
                              sems.at[3])
    for t in range(12, 16):
        sd[t].wait()
    plsc.subcore_barrier()
    pltpu.sync_copy(dacc_sh.at[pl.ds(doff, NPAD // NS)],
                    deg_out.at[c, pl.ds(doff, NPAD // NS)])


def _scat_body(g, src3, dst3, zrows, out, idxs, idxd, rows, acc_sh,
               semi, semg, sems):
    c = lax.axis_index("c")
    s = lax.axis_index("s")
    w = s * NC + c
    off = s * RPT
    cis = pltpu.async_copy(src3.at[w, pl.ds(0, SPH)], idxs, semi)
    cid = pltpu.async_copy(dst3.at[w], idxd, semi)
    pltpu.sync_copy(zrows, acc_sh.at[pl.ds(off, RPT)])
    cis.wait()
    cid.wait()
    plsc.subcore_barrier()

    for p in range(2):
        if p == 1:
            pltpu.sync_copy(src3.at[w, pl.ds(SPH, SPH)], idxs)
        pb = p * SPH

        def outer(j0, carry):
            for b in range(DEPTH2):
                jj = j0 * DEPTH2 + b

                @pl.when(j0 > 0)
                def _():
                    pltpu.make_async_copy(rows.at[b],
                                          acc_sh.at[idxd.at[pb + jj - DEPTH2]],
                                          sems.at[b]).wait()

                pltpu.async_copy(g.at[idxs.at[jj]], rows.at[b], semg.at[b])
                k1 = (b - 1) % DEPTH2

                if b >= 1:
                    pltpu.make_async_copy(g.at[idxs.at[jj - 1]], rows.at[k1],
                                          semg.at[k1]).wait()
                    pltpu.async_copy(rows.at[k1], acc_sh.at[idxd.at[pb + jj - 1]],
                                     sems.at[k1], add=True)
                else:
                    @pl.when(j0 > 0)
                    def _():
                        pltpu.make_async_copy(g.at[idxs.at[jj - 1]],
                                              rows.at[k1], semg.at[k1]).wait()
                        pltpu.async_copy(rows.at[k1],
                                         acc_sh.at[idxd.at[pb + jj - 1]],
                                         sems.at[k1], add=True)
            return carry

        lax.fori_loop(0, SPH // DEPTH2, outer, 0)
        kl = (SPH - 1) % DEPTH2
        pltpu.make_async_copy(g.at[idxs.at[SPH - 1]], rows.at[kl],
                              semg.at[kl]).wait()
        pltpu.async_copy(rows.at[kl], acc_sh.at[idxd.at[pb + SPH - 1]],
                         sems.at[kl], add=True)
        for b in range(DEPTH2):
            j = pb + SPH - DEPTH2 + b
            pltpu.make_async_copy(rows.at[b], acc_sh.at[idxd.at[j]],
                                  sems.at[b]).wait()
    plsc.subcore_barrier()
    pltpu.sync_copy(acc_sh.at[pl.ds(off, RPT)], out.at[c, pl.ds(off, RPT)])


def _sc_emb(fp0, fp1, fp2, fp3, e0, e1, e2, e3, dst3, zrow):
    fn = pl.kernel(
        _emb_body,
        out_type=(jax.ShapeDtypeStruct((4, NPAD, EMB), _f32),
                  jax.ShapeDtypeStruct((NC, NPAD), _f32)),
        mesh=_mesh(),
        scratch_types=[
            pltpu.VMEM((NPAD // NW,), jnp.int32),
            pltpu.VMEM((NPAD // NW,), jnp.int32),
            pltpu.VMEM((NPAD // NW,), jnp.int32),
            pltpu.VMEM((NPAD // NW,), jnp.int32),
            pltpu.VMEM((4, C, EMB), _f32),
            pltpu.VMEM((NCHW, EC), jnp.int32),
            pltpu.VMEM((EC,), _f32),
            pltpu.VMEM_SHARED((NPAD,), _f32),
            pltpu.SemaphoreType.DMA,
            pltpu.SemaphoreType.DMA((4,)),
            pltpu.SemaphoreType.DMA((4,)),
            pltpu.SemaphoreType.DMA,
            pltpu.SemaphoreType.DMA((DEPTH,)),
        ],
    )
    return fn(fp0, fp1, fp2, fp3, e0, e1, e2, e3, dst3, zrow)


def _sc_scatter(g, src3, dst3, zrows):
    fn = pl.kernel(
        _scat_body,
        out_type=jax.ShapeDtypeStruct((NC, NPAD, HID), _f32),
        mesh=_mesh(),
        scratch_types=[
            pltpu.VMEM((SPH, EC), jnp.int32),
            pltpu.VMEM((NCHW, EC), jnp.int32),
            pltpu.VMEM((DEPTH2, EC, HID), _f32),
            pltpu.VMEM_SHARED((NPAD, HID), _f32),
            pltpu.SemaphoreType.DMA,
            pltpu.SemaphoreType.DMA((DEPTH2,)),
            pltpu.SemaphoreType.DMA((DEPTH2,)),
        ],
    )
    return fn(g, src3, dst3, zrows)


def _dinv(deg_ref):
    d = deg_ref[0] + deg_ref[1] + 1.0
    return lax.rsqrt(d)


def _t0_body(embs_ref, pW1_ref, pb1_ref, pW2_ref, pb2_ref, gW0_ref, deg_ref,
             out_ref):
    h = jnp.dot(embs_ref[0], pW1_ref[0], preferred_element_type=_f32)
    for f in range(1, 4):
        h += jnp.dot(embs_ref[f], pW1_ref[f], preferred_element_type=_f32)
    h = jnp.maximum(h + pb1_ref[...], 0.0)
    h = jnp.dot(h, pW2_ref[...], preferred_element_type=_f32) + pb2_ref[...]
    dinv = _dinv(deg_ref)
    out_ref[...] = dinv * jnp.dot(h, gW0_ref[...], preferred_element_type=_f32)


def _tc_proj(embs, pW1r, pb1, pW2, pb2, gW0, degr):
    return pl.pallas_call(
        _t0_body,
        grid=(GRID,),
        in_specs=[
            pl.BlockSpec((4, R, EMB), lambda i: (0, i, 0)),
            pl.BlockSpec((4, EMB, EMB), lambda i: (0, 0, 0)),
            pl.BlockSpec((1, EMB), lambda i: (0, 0)),
            pl.BlockSpec((EMB, EMB), lambda i: (0, 0)),
            pl.BlockSpec((1, EMB), lambda i: (0, 0)),
            pl.BlockSpec((EMB, HID), lambda i: (0, 0)),
            pl.BlockSpec((2, R, 1), lambda i: (0, i, 0)),
        ],
        out_specs=pl.BlockSpec((R, HID), lambda i: (i, 0)),
        out_shape=jax.ShapeDtypeStruct((NPAD, HID), _f32),
    )(embs, pW1r, pb1, pW2, pb2, gW0, degr)


def _comb_body(acc_ref, g_ref, deg_ref, gb_ref, W_ref, out_ref):
    dinv = _dinv(deg_ref)
    x = dinv * (acc_ref[0] + acc_ref[1] + g_ref[...]) + gb_ref[...]
    x = jnp.maximum(x, 0.0)
    out_ref[...] = dinv * jnp.dot(x, W_ref[...], preferred_element_type=_f32)


def _tc_combine(acc, g, degr, gb, W):
    return pl.pallas_call(
        _comb_body,
        grid=(GRID,),
        in_specs=[
            pl.BlockSpec((2, R, HID), lambda i: (0, i, 0)),
            pl.BlockSpec((R, HID), lambda i: (i, 0)),
            pl.BlockSpec((2, R, 1), lambda i: (0, i, 0)),
            pl.BlockSpec((1, HID), lambda i: (0, 0)),
            pl.BlockSpec((HID, HID), lambda i: (0, 0)),
        ],
        out_specs=pl.BlockSpec((R, HID), lambda i: (i, 0)),
        out_shape=jax.ShapeDtypeStruct((NPAD, HID), _f32),
    )(acc, g, degr, gb, W)


def _pool_head_body(acc_ref, g_ref, deg_ref, gb_ref, batch_ref, fp_ref,
                    fpW_ref, fpb_ref, l1a_ref, l1b_ref, l1bias_ref,
                    l2W_ref, l2b_ref, out_ref, pacc):
    i = pl.program_id(0)
    dinv = _dinv(deg_ref)
    x = dinv * (acc_ref[0] + acc_ref[1] + g_ref[...]) + gb_ref[...]
    x = jnp.maximum(x, 0.0)
    bt = batch_ref[...]
    oh = (bt == lax.broadcasted_iota(jnp.int32, (R, B), 1)).astype(_f32)
    cd = (((0,), (0,)), ((), ()))
    part = lax.dot_general(oh, x, cd, preferred_element_type=_f32)

    @pl.when(i == 0)
    def _():
        pacc[...] = part

    @pl.when(i != 0)
    def _():
        pacc[...] += part

    @pl.when(i == GRID - 1)
    def _():
        fpe = jnp.dot(fp_ref[...], fpW_ref[...], preferred_element_type=_f32)
        fpe = fpe + fpb_ref[...]
        z = (jnp.dot(fpe, l1a_ref[...], preferred_element_type=_f32)
             + jnp.dot(pacc[...], l1b_ref[...], preferred_element_type=_f32)
             + l1bias_ref[...])
        z = jnp.maximum(z, 0.0)
        out_ref[...] = (jnp.dot(z, l2W_ref[...], preferred_element_type=_f32)
                        + l2b_ref[...])


def _tc_pool_head(acc, g, degr, gb, batch_p, fingerprint, fpW, fpb,
                  l1Wa, l1Wb, l1b, l2W, l2b):
    return pl.pallas_call(
        _pool_head_body,
        grid=(GRID,),
        in_specs=[
            pl.BlockSpec((2, R, HID), lambda i: (0, i, 0)),
            pl.BlockSpec((R, HID), lambda i: (i, 0)),
            pl.BlockSpec((2, R, 1), lambda i: (0, i, 0)),
            pl.BlockSpec((1, HID), lambda i: (0, 0)),
            pl.BlockSpec((R, 1), lambda i: (i, 0)),
            pl.BlockSpec((B, FP), lambda i: (0, 0)),
            pl.BlockSpec((FP, HID), lambda i: (0, 0)),
            pl.BlockSpec((1, HID), lambda i: (0, 0)),
            pl.BlockSpec((HID, HID // 2), lambda i: (0, 0)),
            pl.BlockSpec((HID, HID // 2), lambda i: (0, 0)),
            pl.BlockSpec((1, HID // 2), lambda i: (0, 0)),
            pl.BlockSpec((HID // 2, 1), lambda i: (0, 0)),
            pl.BlockSpec((1, 1), lambda i: (0, 0)),
        ],
        out_specs=pl.BlockSpec((B, 1), lambda i: (0, 0)),
        out_shape=jax.ShapeDtypeStruct((B, 1), _f32),
        scratch_shapes=[pltpu.VMEM((B, HID), _f32)],
    )(acc, g, degr, gb, batch_p, fingerprint, fpW, fpb,
      l1Wa, l1Wb, l1b, l2W, l2b)


def kernel(f0, f1, f2, f3, edge_index, batch, fingerprint,
           emb0, emb1, emb2, emb3, pW1, pb1, pW2, pb2,
           gW0, gb0, gW1, gb1, gW2, gb2, fpW, fpb, l1W, l1b, l2W, l2b):
    npe = NCHW * EC - EPW
    pad_e = jnp.arange(npe, dtype=jnp.int32) % (NPAD - N) + N
    pad_e = jnp.broadcast_to(pad_e, (NW, npe))
    src3 = jnp.concatenate(
        [edge_index[0].reshape(NW, EPW), pad_e], axis=1).reshape(NW, NCHW, EC)
    dst3 = jnp.concatenate(
        [edge_index[1].reshape(NW, EPW), pad_e], axis=1).reshape(NW, NCHW, EC)

    pad_i = jnp.arange(NPAD - N, dtype=jnp.int32)
    fps = [jnp.concatenate([f.astype(jnp.int32), pad_i])
           for f in (f0, f1, f2, f3)]
    batch_p = jnp.concatenate(
        [batch.astype(jnp.int32), jnp.full((NPAD - N,), B, jnp.int32)]
    ).reshape(NPAD, 1)
    zrow = jnp.zeros((RPT,), _f32)
    zrows = jnp.zeros((RPT, HID), _f32)

    embs, deg = _sc_emb(*fps, emb0, emb1, emb2, emb3, dst3, zrow)
    degr = deg.reshape(NC, NPAD, 1)

    pW1r = pW1.reshape(4, EMB, EMB)
    g0 = _tc_proj(embs, pW1r, pb1.reshape(1, EMB), pW2, pb2.reshape(1, EMB),
                  gW0, degr)
    acc = _sc_scatter(g0, src3, dst3, zrows)
    g1 = _tc_combine(acc, g0, degr, gb0.reshape(1, HID), gW1)
    acc = _sc_scatter(g1, src3, dst3, zrows)
    g2 = _tc_combine(acc, g1, degr, gb1.reshape(1, HID), gW2)
    acc = _sc_scatter(g2, src3, dst3, zrows)
    out = _tc_pool_head(acc, g2, degr, gb2.reshape(1, HID), batch_p,
                        fingerprint, fpW, fpb.reshape(1, HID),
                        l1W[:HID], l1W[HID:], l1b.reshape(1, HID // 2),
                        l2W, l2b.reshape(1, 1))
    return out


# speedup vs baseline: 24.7652x; 1.0022x over previous
import jax
import jax.numpy as jnp
from jax import lax
from jax.experimental import pallas as pl
from jax.experimental.pallas import tpu as pltpu
from jax.experimental.pallas import tpu_sc as plsc

N = 10000
E = 320000
B = 64
EMB = 128
HID = 128
FP = 2048

NPAD = 10240
NC, NS = 2, 16
NW = NC * NS
EPW = E // NW
C = 80
EC = 128
NCHW = 80
SPH = NCHW // 2
HH = HID // 2
RPT = NPAD // NS
DEPTH = 5
DEPTH2 = 2
R = 1024
GRID = NPAD // R

_f32 = jnp.float32


def _mesh():
    return plsc.VectorSubcoreMesh(
        core_axis_name="c", subcore_axis_name="s", num_cores=NC, num_subcores=NS
    )


def _emb_body(f0, f1, f2, f3, t0, t1, t2, t3, dst3, zrow, out, deg_out,
              ix0, ix1, ix2, ix3, rows, idxd, ones_v, dacc_sh,
              semi, semg, sems, semd, semdd):
    c = lax.axis_index("c")
    s = lax.axis_index("s")
    w = s * NC + c
    rbase = w * (NPAD // NW)
    fields = (f0, f1, f2, f3)
    tables = (t0, t1, t2, t3)
    ixs = (ix0, ix1, ix2, ix3)
    doff = s * (NPAD // NS)
    for i in range(EC // 16):
        ones_v[pl.ds(i * 16, 16)] = jnp.ones((16,), _f32)
    di = pltpu.async_copy(dst3.at[w], idxd, semd)
    pltpu.sync_copy(zrow, dacc_sh.at[pl.ds(doff, NPAD // NS)])
    idr = [pltpu.async_copy(fields[f].at[pl.ds(rbase, NPAD // NW)],
                            ixs[f], semi) for f in range(4)]
    di.wait()
    plsc.subcore_barrier()

    def douter(j0, carry):
        for b in range(DEPTH):
            j = j0 * DEPTH + b

            @pl.when(j0 > 0)
            def _():
                pltpu.make_async_copy(ones_v, dacc_sh.at[idxd.at[j]],
                                      semdd.at[b]).wait()

            pltpu.async_copy(ones_v, dacc_sh.at[idxd.at[j]], semdd.at[b],
                             add=True)
        return carry

    lax.fori_loop(0, NCHW // DEPTH, douter, 0)
    for b in range(DEPTH):
        j = NCHW - DEPTH + b
        pltpu.make_async_copy(ones_v, dacc_sh.at[idxd.at[j]],
                              semdd.at[b]).wait()
    for d in idr:
        d.wait()
    gd = [None] * 16
    sd = [None] * 16

    def idx_of(t):
        f, i = t // 4, t % 4
        return f, ixs[f].at[pl.ds(i * C, C)], rbase + i * C

    for t in range(16):
        k = t % 4
        if t >= 4:
            sd[t - 4].wait()
        f, idx, _ = idx_of(t)
        gd[t] = pltpu.async_copy(tables[f].at[idx], rows.at[k], semg.at[k])
        if t >= 1:
            k1 = (t - 1) % 4
            gd[t - 1].wait()
            f1, _, ob = idx_of(t - 1)
            sd[t - 1] = pltpu.async_copy(
                rows.at[k1], out.at[f1, pl.ds(ob, C)], sems.at[k1])
    gd[15].wait()
    f1, _, ob = idx_of(15)
    sd[15] = pltpu.async_copy(rows.at[3], out.at[f1, pl.ds(ob, C)],
                              sems.at[3])
    for t in range(12, 16):
        sd[t].wait()
    plsc.subcore_barrier()
    pltpu.sync_copy(dacc_sh.at[pl.ds(doff, NPAD // NS)],
                    deg_out.at[c, pl.ds(doff, NPAD // NS)])


def _scat_body(g, src3, dst3, zrows, out, idxs, idxd, rows, acc_sh,
               semi, semg, sems):
    c = lax.axis_index("c")
    s = lax.axis_index("s")
    w = s * NC + c
    off = s * RPT
    cis = pltpu.async_copy(src3.at[w, pl.ds(0, SPH)], idxs, semi)
    cid = pltpu.async_copy(dst3.at[w], idxd, semi)
    pltpu.sync_copy(zrows, acc_sh.at[pl.ds(off, RPT)])
    cis.wait()
    cid.wait()
    plsc.subcore_barrier()

    for p in range(2):
        if p == 1:
            pltpu.sync_copy(src3.at[w, pl.ds(SPH, SPH)], idxs)
        pb = p * SPH

        def outer(j0, carry):
            for b in range(DEPTH2):
                jj = j0 * DEPTH2 + b

                @pl.when(j0 > 0)
                def _():
                    pltpu.make_async_copy(rows.at[b],
                                          acc_sh.at[idxd.at[pb + jj - DEPTH2]],
                                          sems.at[b]).wait()

                pltpu.async_copy(g.at[idxs.at[jj]], rows.at[b], semg.at[b])
                k1 = (b - 1) % DEPTH2

                if b >= 1:
                    pltpu.make_async_copy(g.at[idxs.at[jj - 1]], rows.at[k1],
                                          semg.at[k1]).wait()
                    pltpu.async_copy(rows.at[k1], acc_sh.at[idxd.at[pb + jj - 1]],
                                     sems.at[k1], add=True)
                else:
                    @pl.when(j0 > 0)
                    def _():
                        pltpu.make_async_copy(g.at[idxs.at[jj - 1]],
                                              rows.at[k1], semg.at[k1]).wait()
                        pltpu.async_copy(rows.at[k1],
                                         acc_sh.at[idxd.at[pb + jj - 1]],
                                         sems.at[k1], add=True)
            return carry

        lax.fori_loop(0, SPH // DEPTH2, outer, 0)
        kl = (SPH - 1) % DEPTH2
        pltpu.make_async_copy(g.at[idxs.at[SPH - 1]], rows.at[kl],
                              semg.at[kl]).wait()
        pltpu.async_copy(rows.at[kl], acc_sh.at[idxd.at[pb + SPH - 1]],
                         sems.at[kl], add=True)
        for b in range(DEPTH2):
            j = pb + SPH - DEPTH2 + b
            pltpu.make_async_copy(rows.at[b], acc_sh.at[idxd.at[j]],
                                  sems.at[b]).wait()
    plsc.subcore_barrier()
    pltpu.sync_copy(acc_sh.at[pl.ds(off, RPT)], out.at[c, pl.ds(off, RPT)])


def _sc_emb(fp0, fp1, fp2, fp3, e0, e1, e2, e3, dst3, zrow):
    fn = pl.kernel(
        _emb_body,
        out_type=(jax.ShapeDtypeStruct((4, NPAD, EMB), _f32),
                  jax.ShapeDtypeStruct((NC, NPAD), _f32)),
        mesh=_mesh(),
        scratch_types=[
            pltpu.VMEM((NPAD // NW,), jnp.int32),
            pltpu.VMEM((NPAD // NW,), jnp.int32),
            pltpu.VMEM((NPAD // NW,), jnp.int32),
            pltpu.VMEM((NPAD // NW,), jnp.int32),
            pltpu.VMEM((4, C, EMB), _f32),
            pltpu.VMEM((NCHW, EC), jnp.int32),
            pltpu.VMEM((EC,), _f32),
            pltpu.VMEM_SHARED((NPAD,), _f32),
            pltpu.SemaphoreType.DMA,
            pltpu.SemaphoreType.DMA((4,)),
            pltpu.SemaphoreType.DMA((4,)),
            pltpu.SemaphoreType.DMA,
            pltpu.SemaphoreType.DMA((DEPTH,)),
        ],
    )
    return fn(fp0, fp1, fp2, fp3, e0, e1, e2, e3, dst3, zrow)


def _sc_scatter(g, src3, dst3, zrows):
    fn = pl.kernel(
        _scat_body,
        out_type=jax.ShapeDtypeStruct((NC, NPAD, HID), _f32),
        mesh=_mesh(),
        scratch_types=[
            pltpu.VMEM((SPH, EC), jnp.int32),
            pltpu.VMEM((NCHW, EC), jnp.int32),
            pltpu.VMEM((DEPTH2, EC, HID), _f32),
            pltpu.VMEM_SHARED((NPAD, HID), _f32),
            pltpu.SemaphoreType.DMA,
            pltpu.SemaphoreType.DMA((DEPTH2,)),
            pltpu.SemaphoreType.DMA((DEPTH2,)),
        ],
    )
    return fn(g, src3, dst3, zrows)


def _dinv(deg_ref):
    d = deg_ref[0] + deg_ref[1] + 1.0
    return lax.rsqrt(d)


def _t0_body(embs_ref, pW1_ref, pb1_ref, pW2_ref, pb2_ref, gW0_ref, deg_ref,
             out_ref):
    h = jnp.dot(embs_ref[0], pW1_ref[0], preferred_element_type=_f32)
    for f in range(1, 4):
        h += jnp.dot(embs_ref[f], pW1_ref[f], preferred_element_type=_f32)
    h = jnp.maximum(h + pb1_ref[...], 0.0)
    h = jnp.dot(h, pW2_ref[...], preferred_element_type=_f32) + pb2_ref[...]
    dinv = _dinv(deg_ref)
    out_ref[...] = dinv * jnp.dot(h, gW0_ref[...], preferred_element_type=_f32)


def _tc_proj(embs, pW1r, pb1, pW2, pb2, gW0, degr):
    return pl.pallas_call(
        _t0_body,
        grid=(GRID,),
        in_specs=[
            pl.BlockSpec((4, R, EMB), lambda i: (0, i, 0)),
            pl.BlockSpec((4, EMB, EMB), lambda i: (0, 0, 0)),
            pl.BlockSpec((1, EMB), lambda i: (0, 0)),
            pl.BlockSpec((EMB, EMB), lambda i: (0, 0)),
            pl.BlockSpec((1, EMB), lambda i: (0, 0)),
            pl.BlockSpec((EMB, HID), lambda i: (0, 0)),
            pl.BlockSpec((2, R, 1), lambda i: (0, i, 0)),
        ],
        out_specs=pl.BlockSpec((R, HID), lambda i: (i, 0)),
        out_shape=jax.ShapeDtypeStruct((NPAD, HID), _f32),
    )(embs, pW1r, pb1, pW2, pb2, gW0, degr)


def _comb_body(acc_ref, g_ref, deg_ref, gb_ref, W_ref, out_ref):
    dinv = _dinv(deg_ref)
    x = dinv * (acc_ref[0] + acc_ref[1] + g_ref[...]) + gb_ref[...]
    x = jnp.maximum(x, 0.0)
    out_ref[...] = dinv * jnp.dot(x, W_ref[...], preferred_element_type=_f32)


def _tc_combine(acc, g, degr, gb, W):
    return pl.pallas_call(
        _comb_body,
        grid=(GRID,),
        in_specs=[
            pl.BlockSpec((2, R, HID), lambda i: (0, i, 0)),
            pl.BlockSpec((R, HID), lambda i: (i, 0)),
            pl.BlockSpec((2, R, 1), lambda i: (0, i, 0)),
            pl.BlockSpec((1, HID), lambda i: (0, 0)),
            pl.BlockSpec((HID, HID), lambda i: (0, 0)),
        ],
        out_specs=pl.BlockSpec((R, HID), lambda i: (i, 0)),
        out_shape=jax.ShapeDtypeStruct((NPAD, HID), _f32),
    )(acc, g, degr, gb, W)


def _pool_head_body(acc_ref, g_ref, deg_ref, gb_ref, batch_ref, fp_ref,
                    fpW_ref, fpb_ref, l1a_ref, l1b_ref, l1bias_ref,
                    l2W_ref, l2b_ref, out_ref, pacc):
    i = pl.program_id(0)
    dinv = _dinv(deg_ref)
    x = dinv * (acc_ref[0] + acc_ref[1] + g_ref[...]) + gb_ref[...]
    x = jnp.maximum(x, 0.0)
    bt = batch_ref[...]
    oh = (bt == lax.broadcasted_iota(jnp.int32, (R, B), 1)).astype(_f32)
    cd = (((0,), (0,)), ((), ()))
    part = lax.dot_general(oh, x, cd, preferred_element_type=_f32)

    @pl.when(i == 0)
    def _():
        pacc[...] = part

    @pl.when(i != 0)
    def _():
        pacc[...] += part

    @pl.when(i == GRID - 1)
    def _():
        fpe = jnp.dot(fp_ref[...], fpW_ref[...], preferred_element_type=_f32)
        fpe = fpe + fpb_ref[...]
        z = (jnp.dot(fpe, l1a_ref[...], preferred_element_type=_f32)
             + jnp.dot(pacc[...], l1b_ref[...], preferred_element_type=_f32)
             + l1bias_ref[...])
        z = jnp.maximum(z, 0.0)
        out_ref[...] = (jnp.dot(z, l2W_ref[...], preferred_element_type=_f32)
                        + l2b_ref[...])


def _tc_pool_head(acc, g, degr, gb, batch_p, fingerprint, fpW, fpb,
                  l1Wa, l1Wb, l1b, l2W, l2b):
    return pl.pallas_call(
        _pool_head_body,
        grid=(GRID,),
        in_specs=[
            pl.BlockSpec((2, R, HID), lambda i: (0, i, 0)),
            pl.BlockSpec((R, HID), lambda i: (i, 0)),
            pl.BlockSpec((2, R, 1), lambda i: (0, i, 0)),
            pl.BlockSpec((1, HID), lambda i: (0, 0)),
            pl.BlockSpec((R, 1), lambda i: (i, 0)),
            pl.BlockSpec((B, FP), lambda i: (0, 0)),
            pl.BlockSpec((FP, HID), lambda i: (0, 0)),
            pl.BlockSpec((1, HID), lambda i: (0, 0)),
            pl.BlockSpec((HID, HID // 2), lambda i: (0, 0)),
            pl.BlockSpec((HID, HID // 2), lambda i: (0, 0)),
            pl.BlockSpec((1, HID // 2), lambda i: (0, 0)),
            pl.BlockSpec((HID // 2, 1), lambda i: (0, 0)),
            pl.BlockSpec((1, 1), lambda i: (0, 0)),
        ],
        out_specs=pl.BlockSpec((B, 1), lambda i: (0, 0)),
        out_shape=jax.ShapeDtypeStruct((B, 1), _f32),
        scratch_shapes=[pltpu.VMEM((B, HID), _f32)],
    )(acc, g, degr, gb, batch_p, fingerprint, fpW, fpb,
      l1Wa, l1Wb, l1b, l2W, l2b)


def kernel(f0, f1, f2, f3, edge_index, batch, fingerprint,
           emb0, emb1, emb2, emb3, pW1, pb1, pW2, pb2,
           gW0, gb0, gW1, gb1, gW2, gb2, fpW, fpb, l1W, l1b, l2W, l2b):
    npe = NCHW * EC - EPW
    pad_e = jnp.arange(npe, dtype=jnp.int32) % (NPAD - N) + N
    pad_e = jnp.broadcast_to(pad_e, (NW, npe))
    src3 = jnp.concatenate(
        [edge_index[0].reshape(NW, EPW), pad_e], axis=1).reshape(NW, NCHW, EC)
    dst3 = jnp.concatenate(
        [edge_index[1].reshape(NW, EPW), pad_e], axis=1).reshape(NW, NCHW, EC)

    pad_i = jnp.arange(NPAD - N, dtype=jnp.int32)
    fps = [jnp.concatenate([f.astype(jnp.int32), pad_i])
           for f in (f0, f1, f2, f3)]
    batch_p = jnp.concatenate(
        [batch.astype(jnp.int32), jnp.full((NPAD - N,), B, jnp.int32)]
    ).reshape(NPAD, 1)
    zrow = jnp.zeros((RPT,), _f32)
    zrows = jnp.zeros((RPT, HID), _f32)

    embs, deg = _sc_emb(*fps, emb0, emb1, emb2, emb3, dst3, zrow)
    degr = deg.reshape(NC, NPAD, 1)

    pW1r = pW1.reshape(4, EMB, EMB)
    g0 = _tc_proj(embs, pW1r, pb1.reshape(1, EMB), pW2, pb2.reshape(1, EMB),
                  gW0, degr)
    acc = _sc_scatter(g0, src3, dst3, zrows)
    g1 = _tc_combine(acc, g0, degr, gb0.reshape(1, HID), gW1)
    acc = _sc_scatter(g1, src3, dst3, zrows)
    g2 = _tc_combine(acc, g1, degr, gb1.reshape(1, HID), gW2)
    acc = _sc_scatter(g2, src3, dst3, zrows)
    out = _tc_pool_head(acc, g2, degr, gb2.reshape(1, HID), batch_p,
                        fingerprint, fpW, fpb.reshape(1, HID),
                        l1W[:HID], l1W[HID:], l1b.reshape(1, HID // 2),
                        l2W, l2b.reshape(1, 1))
    return out
